# Initial kernel scaffold; baseline (speedup 1.0000x reference)
import jax
import jax.numpy as jnp
from jax.experimental import pallas as pl


def kernel(users_feature, items_feature, bundles_feature, IL_user_hyper, IL_item_hyper, BL_user_hyper, BL_bundle_hyper, ui_src, ui_dst, ub_src, ub_dst, bi_src, bi_dst, users, bundles):
    raise NotImplementedError("write your pallas kernel here")


# speedup vs baseline: 1.0000x; 1.0000x over previous
import jax
import jax.numpy as jnp
from jax.experimental import pallas as pl

NUM_LAYERS = 2


def _laplace_norm(rows, cols, n):
    deg = jax.ops.segment_sum(jnp.ones(rows.shape, jnp.float32), rows, num_segments=n)
    d_inv = 1.0 / (jnp.sqrt(deg) + 1e-8)
    return d_inv[rows] * d_inv[cols]


def _one_propagate(feat_a, feat_b, src, dst, num_layers=NUM_LAYERS):
    na = feat_a.shape[0]
    nb = feat_b.shape[0]
    n = na + nb
    rows = jnp.concatenate([src, dst + na])
    cols = jnp.concatenate([dst + na, src])
    w = _laplace_norm(rows, cols, n)
    x = jnp.concatenate([feat_a, feat_b], axis=0)
    acc = x
    for _ in range(num_layers):
        x = jax.ops.segment_sum(x[cols] * w[:, None], rows, num_segments=n)
        acc = acc + x
    acc = acc / (num_layers + 1)
    return acc[:na], acc[na:]


def _hyper_conv(x, hmat):
    h = jax.nn.leaky_relu(x @ hmat, negative_slope=0.05)
    return jax.nn.leaky_relu(h @ hmat.T, negative_slope=0.05)


def _score_kernel(u_ref, b_ref, o_ref):
    u = u_ref[...]
    br = b_ref[...]
    p0 = jnp.sum(u * br[:, :64], axis=1, keepdims=True)
    p1 = jnp.sum(u * br[:, 64:], axis=1, keepdims=True)
    o_ref[...] = jnp.concatenate([p0, p1], axis=1)


def kernel(users_feature, items_feature, bundles_feature, IL_user_hyper, IL_item_hyper, BL_user_hyper, BL_bundle_hyper, ui_src, ui_dst, ub_src, ub_dst, bi_src, bi_dst, users, bundles):
    IL_users, IL_items = _one_propagate(users_feature, items_feature, ui_src, ui_dst)
    BL_users, BL_bundles = _one_propagate(users_feature, bundles_feature, ub_src, ub_dst)
    IL_users = IL_users + _hyper_conv(IL_users, IL_user_hyper)
    IL_items = IL_items + _hyper_conv(IL_items, IL_item_hyper)
    BL_users = BL_users + _hyper_conv(BL_users, BL_user_hyper)
    BL_bundles = BL_bundles + _hyper_conv(BL_bundles, BL_bundle_hyper)
    nb = bundles_feature.shape[0]
    deg_b = jax.ops.segment_sum(jnp.ones(bi_src.shape, jnp.float32), bi_src, num_segments=nb)
    IL_bundles = jax.ops.segment_sum(IL_items[bi_dst], bi_src, num_segments=nb) / (deg_b[:, None] + 1e-8)
    users_rep = IL_users[users] + BL_users[users]
    bundles_rep = (IL_bundles[bundles] + BL_bundles[bundles]).reshape(-1, 128)
    B = users_rep.shape[0]
    pred = pl.pallas_call(
        _score_kernel,
        out_shape=jax.ShapeDtypeStruct((B, 2), jnp.float32),
    )(users_rep, bundles_rep)
    return pred


# speedup vs baseline: 8.2361x; 8.2358x over previous
import functools

import jax
import jax.numpy as jnp
from jax import lax
from jax.experimental import pallas as pl
from jax.experimental.pallas import tpu as pltpu
from jax.experimental.pallas import tpu_sc as plsc

NU, NI, NB, D, H = 50000, 50000, 20000, 64, 128
LANES = 16
CHUNK = 128
NTILE = 16

R50, A50 = 26624, 28672
R20, A20 = 10240, 12288
EP8 = 800768
EPB = 641024

_mesh = plsc.VectorSubcoreMesh(core_axis_name="c", subcore_axis_name="s")


def _make_spmm(n_gather, e_pad, r, acc_rows):
    ept = e_pad // NTILE
    n_ch = ept // CHUNK
    rpt = r // NTILE
    zpt = acc_rows // NTILE // CHUNK
    dmask = acc_rows - r - 1

    @functools.partial(
        pl.kernel,
        out_type=jax.ShapeDtypeStruct((2 * r, D), jnp.float32),
        mesh=_mesh,
        compiler_params=pltpu.CompilerParams(use_tc_tiling_on_sc=False),
        scratch_types=[
            pltpu.VMEM_SHARED((acc_rows, D), jnp.float32),
            pltpu.VMEM((CHUNK,), jnp.int32),
            pltpu.VMEM((CHUNK,), jnp.int32),
            pltpu.VMEM((CHUNK,), jnp.int32),
            pltpu.VMEM((CHUNK, D), jnp.float32),
            pltpu.SemaphoreType.DMA,
        ],
    )
    def spmm(y_hbm, gidx_hbm, didx_hbm, out_hbm, acc, gbuf, dbuf, sbuf, rbuf, sem):
        c = lax.axis_index("c")
        s = lax.axis_index("s")
        base = c * r
        iota = lax.iota(jnp.int32, LANES)

        def zrow(i, carry):
            for k in range(D // LANES):
                rbuf[i, pl.ds(k * LANES, LANES)] = jnp.zeros((LANES,), jnp.float32)
            return carry

        lax.fori_loop(0, CHUNK, zrow, 0)
        for z in range(zpt):
            pltpu.sync_copy(rbuf, acc.at[pl.ds(s * (acc_rows // NTILE) + z * CHUNK, CHUNK)])
        plsc.subcore_barrier()

        ebase = s * ept

        def body(ch, carry):
            off = ebase + ch * CHUNK
            pltpu.sync_copy(gidx_hbm.at[pl.ds(off, CHUNK)], gbuf)
            pltpu.sync_copy(didx_hbm.at[pl.ds(off, CHUNK)], dbuf)
            for j in range(CHUNK // LANES):
                dv = dbuf[pl.ds(j * LANES, LANES)]
                loc = dv - base
                ok = (loc >= 0) & (loc < r)
                spread = (iota + (ch * CHUNK + j * LANES)) & dmask
                sbuf[pl.ds(j * LANES, LANES)] = jnp.where(ok, loc, r + spread)
            pltpu.async_copy(y_hbm.at[gbuf], rbuf, sem).wait()
            pltpu.sync_copy(rbuf, acc.at[sbuf], add=True)
            return carry

        lax.fori_loop(0, n_ch, body, 0)
        plsc.subcore_barrier()
        pltpu.sync_copy(acc.at[pl.ds(s * rpt, rpt)], out_hbm.at[pl.ds(base + s * rpt, rpt)])

    return spmm


_HIST_CFG = [(EP8, R50, A50), (EP8, R50, A50), (EP8, R50, A50),
             (EP8, R20, A20), (EPB, R20, A20)]


@functools.partial(
    pl.kernel,
    out_type=tuple(jax.ShapeDtypeStruct((2 * r,), jnp.float32) for (_, r, _) in _HIST_CFG),
    mesh=_mesh,
    compiler_params=pltpu.CompilerParams(use_tc_tiling_on_sc=False),
    scratch_types=[
        pltpu.VMEM_SHARED((A50,), jnp.float32),
        pltpu.VMEM((CHUNK,), jnp.int32),
        pltpu.VMEM((CHUNK,), jnp.int32),
        pltpu.VMEM((CHUNK,), jnp.float32),
        pltpu.VMEM((CHUNK,), jnp.float32),
    ],
)
def _hist5(i1, i2, i3, i4, i5, o1, o2, o3, o4, o5, acc, dbuf, sbuf, ones, zbuf):
    c = lax.axis_index("c")
    s = lax.axis_index("s")
    iota = lax.iota(jnp.int32, LANES)
    for j in range(CHUNK // LANES):
        ones[pl.ds(j * LANES, LANES)] = jnp.full((LANES,), 1.0, jnp.float32)
        zbuf[pl.ds(j * LANES, LANES)] = jnp.zeros((LANES,), jnp.float32)

    for idx_hbm, out_hbm, (e_pad, r, acc_rows) in zip(
            (i1, i2, i3, i4, i5), (o1, o2, o3, o4, o5), _HIST_CFG):
        base = c * r
        dmask = acc_rows - r - 1
        ept = e_pad // NTILE
        n_ch = ept // CHUNK
        rpt = r // NTILE
        for z in range(acc_rows // NTILE // CHUNK):
            pltpu.sync_copy(zbuf, acc.at[pl.ds(s * (acc_rows // NTILE) + z * CHUNK, CHUNK)])
        plsc.subcore_barrier()

        ebase = s * ept

        def body(ch, carry):
            off = ebase + ch * CHUNK
            pltpu.sync_copy(idx_hbm.at[pl.ds(off, CHUNK)], dbuf)
            for j in range(CHUNK // LANES):
                dv = dbuf[pl.ds(j * LANES, LANES)]
                loc = dv - base
                ok = (loc >= 0) & (loc < r)
                spread = (iota + (ch * CHUNK + j * LANES)) & dmask
                sbuf[pl.ds(j * LANES, LANES)] = jnp.where(ok, loc, r + spread)
            pltpu.sync_copy(ones, acc.at[sbuf], add=True)
            return carry

        lax.fori_loop(0, n_ch, body, 0)
        plsc.subcore_barrier()
        pltpu.sync_copy(acc.at[pl.ds(s * rpt, rpt)], out_hbm.at[pl.ds(base + s * rpt, rpt)])
        plsc.subcore_barrier()


@functools.partial(
    pl.kernel,
    out_type=(jax.ShapeDtypeStruct((4096, D), jnp.float32),
              jax.ShapeDtypeStruct((8192, D), jnp.float32)),
    mesh=_mesh,
    compiler_params=pltpu.CompilerParams(use_tc_tiling_on_sc=False),
    scratch_types=[
        pltpu.VMEM((CHUNK,), jnp.int32),
        pltpu.VMEM((CHUNK, D), jnp.float32),
        pltpu.VMEM((CHUNK, D), jnp.float32),
        pltpu.SemaphoreType.DMA,
    ],
)
def _batch_gather(ilu, blu, uidx, ilb, blb, bidx, u_out, b_out,
                  ibuf, r1, r2, sem):
    c = lax.axis_index("c")
    s = lax.axis_index("s")
    wid = s * 2 + c

    def addrows(i, carry):
        for k in range(D // LANES):
            sl = pl.ds(k * LANES, LANES)
            r1[i, sl] = r1[i, sl] + r2[i, sl]
        return carry

    off = wid * CHUNK
    pltpu.sync_copy(uidx.at[pl.ds(off, CHUNK)], ibuf)
    pltpu.async_copy(ilu.at[ibuf], r1, sem).wait()
    pltpu.async_copy(blu.at[ibuf], r2, sem).wait()
    lax.fori_loop(0, CHUNK, addrows, 0)
    pltpu.sync_copy(r1, u_out.at[pl.ds(off, CHUNK)])

    for cc in range(2):
        off = wid * 2 * CHUNK + cc * CHUNK
        pltpu.sync_copy(bidx.at[pl.ds(off, CHUNK)], ibuf)
        pltpu.async_copy(ilb.at[ibuf], r1, sem).wait()
        pltpu.async_copy(blb.at[ibuf], r2, sem).wait()
        lax.fori_loop(0, CHUNK, addrows, 0)
        pltpu.sync_copy(r1, b_out.at[pl.ds(off, CHUNK)])


def _dinv_body(x_ref, o_ref):
    o_ref[...] = 1.0 / (jnp.sqrt(x_ref[...]) + 1e-8)


def _invdeg_body(x_ref, o_ref):
    o_ref[...] = 1.0 / (x_ref[...] + 1e-8)


def _ew(body, x):
    n = x.shape[0]
    x2 = x.reshape(n // 128, 128)
    out = pl.pallas_call(body, out_shape=jax.ShapeDtypeStruct(x2.shape, jnp.float32))(x2)
    return out.reshape(n)


def _rowscale_body(x_ref, s_ref, o_ref, *, square):
    sc = s_ref[...]
    if square:
        sc = sc * sc
    o_ref[...] = x_ref[...] * sc


def _rowscale(x, s, out_rows, square=False):
    n = x.shape[0]
    blk = 2000 if n % 2000 == 0 else 2048
    grid = n // blk
    return pl.pallas_call(
        functools.partial(_rowscale_body, square=square),
        grid=(grid,),
        in_specs=[pl.BlockSpec((blk, D), lambda i: (i, 0)),
                  pl.BlockSpec((blk, 1), lambda i: (i, 0))],
        out_specs=pl.BlockSpec((blk, D), lambda i: (i, 0)),
        out_shape=jax.ShapeDtypeStruct((out_rows, D), jnp.float32),
    )(x, s[:, None])


def _lrelu(x):
    return jnp.where(x > 0, x, 0.05 * x)


def _fuse_body(x_ref, r1_ref, r2_ref, s_ref, h_ref, o_ref):
    acc = (x_ref[...] + s_ref[...] * (r1_ref[...] + r2_ref[...])) * (1.0 / 3.0)
    t = _lrelu(jnp.dot(acc, h_ref[...], preferred_element_type=jnp.float32))
    t2 = _lrelu(lax.dot_general(t, h_ref[...], (((1,), (1,)), ((), ())),
                                preferred_element_type=jnp.float32))
    o_ref[...] = acc + t2


def _fuse(x, r1, r2, s, h, out_rows):
    n = x.shape[0]
    blk = 2000
    grid = n // blk
    return pl.pallas_call(
        _fuse_body,
        grid=(grid,),
        in_specs=[pl.BlockSpec((blk, D), lambda i: (i, 0)),
                  pl.BlockSpec((blk, D), lambda i: (i, 0)),
                  pl.BlockSpec((blk, D), lambda i: (i, 0)),
                  pl.BlockSpec((blk, 1), lambda i: (i, 0)),
                  pl.BlockSpec((D, H), lambda i: (0, 0))],
        out_specs=pl.BlockSpec((blk, D), lambda i: (i, 0)),
        out_shape=jax.ShapeDtypeStruct((out_rows, D), jnp.float32),
    )(x, r1, r2, s[:, None], h)


def _score_body(u_ref, b_ref, o_ref):
    u = u_ref[...]
    br = b_ref[...]
    p0 = jnp.sum(u * br[:, :D], axis=1, keepdims=True)
    p1 = jnp.sum(u * br[:, D:], axis=1, keepdims=True)
    o_ref[...] = jnp.concatenate([p0, p1], axis=1)


def _pad_scatter(idx, e_pad):
    pad = jnp.full((e_pad - idx.shape[0],), 100_000_000, jnp.int32)
    return jnp.concatenate([idx.astype(jnp.int32), pad])


def _pad_gather(idx, e_pad, n):
    pad = jnp.arange(e_pad - idx.shape[0], dtype=jnp.int32) % n
    return jnp.concatenate([idx.astype(jnp.int32), pad])


_spmm_a = _make_spmm(2 * R50, EP8, R50, A50)
_spmm_b = _make_spmm(2 * R20, EP8, R50, A50)
_spmm_c = _make_spmm(2 * R50, EP8, R20, A20)
_spmm_d = _make_spmm(2 * R50, EPB, R20, A20)


def kernel(users_feature, items_feature, bundles_feature, IL_user_hyper, IL_item_hyper, BL_user_hyper, BL_bundle_hyper, ui_src, ui_dst, ub_src, ub_dst, bi_src, bi_dst, users, bundles):
    ui_s_sc = _pad_scatter(ui_src, EP8)
    ui_s_ga = _pad_gather(ui_src, EP8, NU)
    ui_d_sc = _pad_scatter(ui_dst, EP8)
    ui_d_ga = _pad_gather(ui_dst, EP8, NI)
    ub_s_sc = _pad_scatter(ub_src, EP8)
    ub_s_ga = _pad_gather(ub_src, EP8, NU)
    ub_d_sc = _pad_scatter(ub_dst, EP8)
    ub_d_ga = _pad_gather(ub_dst, EP8, NB)
    bi_s_sc = _pad_scatter(bi_src, EPB)
    bi_d_ga = _pad_gather(bi_dst, EPB, NI)

    deg_ui_u, deg_ui_i, deg_ub_u, deg_ub_b, deg_bi = _hist5(
        ui_s_sc, ui_d_sc, ub_s_sc, ub_d_sc, bi_s_sc)

    ra_ui = _ew(_dinv_body, deg_ui_u)
    rb_ui = _ew(_dinv_body, deg_ui_i)
    ra_ub = _ew(_dinv_body, deg_ub_u)
    rb_ub = _ew(_dinv_body, deg_ub_b)
    inv_bi = _ew(_invdeg_body, deg_bi)

    yu_ui = _rowscale(users_feature, ra_ui[:NU], 2 * R50)
    yi_ui = _rowscale(items_feature, rb_ui[:NI], 2 * R50)
    yu_ub = _rowscale(users_feature, ra_ub[:NU], 2 * R50)
    yb_ub = _rowscale(bundles_feature, rb_ub[:NB], 2 * R20)

    raw_u1_il = _spmm_a(yi_ui, ui_d_ga, ui_s_sc)
    raw_i1 = _spmm_a(yu_ui, ui_s_ga, ui_d_sc)
    raw_u1_bl = _spmm_b(yb_ub, ub_d_ga, ub_s_sc)
    raw_b1 = _spmm_c(yu_ub, ub_s_ga, ub_d_sc)

    y2_i = _rowscale(raw_i1, rb_ui, 2 * R50, square=True)
    y2_u_il = _rowscale(raw_u1_il, ra_ui, 2 * R50, square=True)
    y2_b = _rowscale(raw_b1, rb_ub, 2 * R20, square=True)
    y2_u_bl = _rowscale(raw_u1_bl, ra_ub, 2 * R50, square=True)

    raw_u2_il = _spmm_a(y2_i, ui_d_ga, ui_s_sc)
    raw_i2 = _spmm_a(y2_u_il, ui_s_ga, ui_d_sc)
    raw_u2_bl = _spmm_b(y2_b, ub_d_ga, ub_s_sc)
    raw_b2 = _spmm_c(y2_u_bl, ub_s_ga, ub_d_sc)

    IL_users_f = _fuse(users_feature, raw_u1_il[:NU], raw_u2_il[:NU], ra_ui[:NU], IL_user_hyper, NU)
    IL_items_f = _fuse(items_feature, raw_i1[:NI], raw_i2[:NI], rb_ui[:NI], IL_item_hyper, 2 * R50)
    BL_users_f = _fuse(users_feature, raw_u1_bl[:NU], raw_u2_bl[:NU], ra_ub[:NU], BL_user_hyper, NU)
    BL_bundles_f = _fuse(bundles_feature, raw_b1[:NB], raw_b2[:NB], rb_ub[:NB], BL_bundle_hyper, NB)

    raw_bi = _spmm_d(IL_items_f, bi_d_ga, bi_s_sc)
    il_bundles = _rowscale(raw_bi, inv_bi, 2 * R20)

    u_rows, b_rows = _batch_gather(
        IL_users_f, BL_users_f, users.astype(jnp.int32),
        il_bundles, BL_bundles_f, bundles.reshape(-1).astype(jnp.int32))

    pred = pl.pallas_call(
        _score_body,
        out_shape=jax.ShapeDtypeStruct((4096, 2), jnp.float32),
    )(u_rows, b_rows.reshape(4096, 2 * D))
    return pred


# speedup vs baseline: 18.2563x; 2.2166x over previous
import functools

import jax
import jax.numpy as jnp
from jax import lax
from jax.experimental import pallas as pl
from jax.experimental.pallas import tpu as pltpu
from jax.experimental.pallas import tpu_sc as plsc

NU, NI, NB, D, H = 50000, 50000, 20000, 64, 128
LANES = 16
CHUNK = 128
NTILE = 16

R50, A50 = 25600, 26624
R20, A20 = 10240, 12288
NBUF = 8
EP8 = 819200
EPB = 655360

_mesh = plsc.VectorSubcoreMesh(core_axis_name="c", subcore_axis_name="s")


def _make_spmm(n_gather, e_pad, r, acc_rows, nbuf):
    ept = e_pad // NTILE
    n_ch = ept // CHUNK
    rpt = r // NTILE
    zpt = acc_rows // NTILE // CHUNK
    dmask = acc_rows - r - 1

    n_body = n_ch // nbuf
    half = nbuf // 2

    @functools.partial(
        pl.kernel,
        out_type=jax.ShapeDtypeStruct((2 * r, D), jnp.float32),
        mesh=_mesh,
        compiler_params=pltpu.CompilerParams(use_tc_tiling_on_sc=False),
        scratch_types=[
            pltpu.VMEM_SHARED((acc_rows, D), jnp.float32),
            pltpu.VMEM((nbuf, CHUNK), jnp.int32),
            pltpu.VMEM((nbuf, CHUNK), jnp.int32),
            pltpu.VMEM((nbuf, CHUNK), jnp.int32),
            pltpu.VMEM((nbuf, CHUNK, D), jnp.float32),
            pltpu.SemaphoreType.DMA,
            pltpu.SemaphoreType.DMA,
        ],
    )
    def spmm(y_hbm, gidx_hbm, didx_hbm, out_hbm, acc, gblk, dblk, sblk, rbuf, gsem, ssem):
        c = lax.axis_index("c")
        s = lax.axis_index("s")
        base = c * r
        iota = lax.iota(jnp.int32, LANES)

        def zrow(i, carry):
            for k in range(D // LANES):
                rbuf[0, i, pl.ds(k * LANES, LANES)] = jnp.zeros((LANES,), jnp.float32)
            return carry

        lax.fori_loop(0, CHUNK, zrow, 0)
        for z in range(zpt):
            pltpu.sync_copy(rbuf.at[0],
                            acc.at[pl.ds(s * (acc_rows // NTILE) + z * CHUNK, CHUNK)])
        plsc.subcore_barrier()

        rbase = s * n_ch

        def body(g, carry):
            row0 = rbase + g * nbuf
            d1 = pltpu.async_copy(gidx_hbm.at[pl.ds(row0, nbuf)], gblk, gsem)
            d2 = pltpu.async_copy(didx_hbm.at[pl.ds(row0, nbuf)], dblk, gsem)
            d1.wait()
            d2.wait()
            gd = [pltpu.async_copy(y_hbm.at[gblk.at[b]], rbuf.at[b], gsem)
                  for b in range(nbuf)]
            for b in range(nbuf):
                for j in range(CHUNK // LANES):
                    dv = dblk[b, pl.ds(j * LANES, LANES)]
                    loc = dv - base
                    ok = (loc >= 0) & (loc < r)
                    spread = (iota + ((g * nbuf + b) * CHUNK + j * LANES)) & dmask
                    sblk[b, pl.ds(j * LANES, LANES)] = jnp.where(ok, loc, r + spread)
            sd = []
            for b in range(half):
                gd[b].wait()
                sd.append(pltpu.async_copy(rbuf.at[b], acc.at[sblk.at[b]], ssem, add=True))
            for b in range(half, nbuf):
                gd[b].wait()
                sd.append(pltpu.async_copy(rbuf.at[b], acc.at[sblk.at[b]], ssem, add=True))
            for d in sd:
                d.wait()
            return carry

        lax.fori_loop(0, n_body, body, 0)
        plsc.subcore_barrier()
        pltpu.sync_copy(acc.at[pl.ds(s * rpt, rpt)], out_hbm.at[pl.ds(base + s * rpt, rpt)])

    return spmm


_HIST_CFG = [(EP8, R50, A50), (EP8, R50, A50), (EP8, R50, A50),
             (EP8, R20, A20), (EPB, R20, A20)]


@functools.partial(
    pl.kernel,
    out_type=tuple(jax.ShapeDtypeStruct((2 * r,), jnp.float32) for (_, r, _) in _HIST_CFG),
    mesh=_mesh,
    compiler_params=pltpu.CompilerParams(use_tc_tiling_on_sc=False),
    scratch_types=[
        pltpu.VMEM_SHARED((A50,), jnp.float32),
        pltpu.VMEM((NBUF, CHUNK), jnp.int32),
        pltpu.VMEM((NBUF, CHUNK), jnp.int32),
        pltpu.VMEM((CHUNK,), jnp.float32),
        pltpu.VMEM((CHUNK,), jnp.float32),
        pltpu.SemaphoreType.DMA,
        pltpu.SemaphoreType.DMA,
    ],
)
def _hist5(i1, i2, i3, i4, i5, o1, o2, o3, o4, o5, acc, dblk, sblk, ones, zbuf, gsem, ssem):
    c = lax.axis_index("c")
    s = lax.axis_index("s")
    iota = lax.iota(jnp.int32, LANES)
    for j in range(CHUNK // LANES):
        ones[pl.ds(j * LANES, LANES)] = jnp.full((LANES,), 1.0, jnp.float32)
        zbuf[pl.ds(j * LANES, LANES)] = jnp.zeros((LANES,), jnp.float32)

    for idx_hbm, out_hbm, (e_pad, r, acc_rows) in zip(
            (i1, i2, i3, i4, i5), (o1, o2, o3, o4, o5), _HIST_CFG):
        base = c * r
        dmask = acc_rows - r - 1
        n_ch = e_pad // NTILE // CHUNK
        rpt = r // NTILE
        for z in range(acc_rows // NTILE // CHUNK):
            pltpu.sync_copy(zbuf, acc.at[pl.ds(s * (acc_rows // NTILE) + z * CHUNK, CHUNK)])
        plsc.subcore_barrier()

        rbase = s * n_ch

        def body(g, carry):
            row0 = rbase + g * NBUF
            pltpu.async_copy(idx_hbm.at[pl.ds(row0, NBUF)], dblk, gsem).wait()
            for b in range(NBUF):
                for j in range(CHUNK // LANES):
                    dv = dblk[b, pl.ds(j * LANES, LANES)]
                    loc = dv - base
                    ok = (loc >= 0) & (loc < r)
                    spread = (iota + ((g * NBUF + b) * CHUNK + j * LANES)) & dmask
                    sblk[b, pl.ds(j * LANES, LANES)] = jnp.where(ok, loc, r + spread)
            sd = [pltpu.async_copy(ones, acc.at[sblk.at[b]], ssem, add=True)
                  for b in range(NBUF)]
            for d in sd:
                d.wait()
            return carry

        lax.fori_loop(0, n_ch // NBUF, body, 0)
        plsc.subcore_barrier()
        pltpu.sync_copy(acc.at[pl.ds(s * rpt, rpt)], out_hbm.at[pl.ds(base + s * rpt, rpt)])
        plsc.subcore_barrier()


@functools.partial(
    pl.kernel,
    out_type=(jax.ShapeDtypeStruct((4096, D), jnp.float32),
              jax.ShapeDtypeStruct((8192, D), jnp.float32)),
    mesh=_mesh,
    compiler_params=pltpu.CompilerParams(use_tc_tiling_on_sc=False),
    scratch_types=[
        pltpu.VMEM((CHUNK,), jnp.int32),
        pltpu.VMEM((CHUNK, D), jnp.float32),
        pltpu.VMEM((CHUNK, D), jnp.float32),
        pltpu.SemaphoreType.DMA,
    ],
)
def _batch_gather(ilu, blu, uidx, ilb, blb, bidx, u_out, b_out,
                  ibuf, r1, r2, sem):
    c = lax.axis_index("c")
    s = lax.axis_index("s")
    wid = s * 2 + c

    def addrows(i, carry):
        for k in range(D // LANES):
            sl = pl.ds(k * LANES, LANES)
            r1[i, sl] = r1[i, sl] + r2[i, sl]
        return carry

    off = wid * CHUNK
    pltpu.sync_copy(uidx.at[pl.ds(off, CHUNK)], ibuf)
    pltpu.async_copy(ilu.at[ibuf], r1, sem).wait()
    pltpu.async_copy(blu.at[ibuf], r2, sem).wait()
    lax.fori_loop(0, CHUNK, addrows, 0)
    pltpu.sync_copy(r1, u_out.at[pl.ds(off, CHUNK)])

    for cc in range(2):
        off = wid * 2 * CHUNK + cc * CHUNK
        pltpu.sync_copy(bidx.at[pl.ds(off, CHUNK)], ibuf)
        pltpu.async_copy(ilb.at[ibuf], r1, sem).wait()
        pltpu.async_copy(blb.at[ibuf], r2, sem).wait()
        lax.fori_loop(0, CHUNK, addrows, 0)
        pltpu.sync_copy(r1, b_out.at[pl.ds(off, CHUNK)])


def _dinv_body(x_ref, o_ref):
    o_ref[...] = 1.0 / (jnp.sqrt(x_ref[...]) + 1e-8)


def _invdeg_body(x_ref, o_ref):
    o_ref[...] = 1.0 / (x_ref[...] + 1e-8)


def _ew(body, x):
    n = x.shape[0]
    x2 = x.reshape(n // 128, 128)
    out = pl.pallas_call(body, out_shape=jax.ShapeDtypeStruct(x2.shape, jnp.float32))(x2)
    return out.reshape(n)


def _rowscale_body(x_ref, s_ref, o_ref, *, square):
    sc = s_ref[...]
    if square:
        sc = sc * sc
    o_ref[...] = x_ref[...] * sc


def _rowscale(x, s, out_rows, square=False):
    n = x.shape[0]
    blk = 2000 if n % 2000 == 0 else 2048
    grid = n // blk
    return pl.pallas_call(
        functools.partial(_rowscale_body, square=square),
        grid=(grid,),
        in_specs=[pl.BlockSpec((blk, D), lambda i: (i, 0)),
                  pl.BlockSpec((blk, 1), lambda i: (i, 0))],
        out_specs=pl.BlockSpec((blk, D), lambda i: (i, 0)),
        out_shape=jax.ShapeDtypeStruct((out_rows, D), jnp.float32),
    )(x, s[:, None])


def _lrelu(x):
    return jnp.where(x > 0, x, 0.05 * x)


def _fuse_body(x_ref, r1_ref, r2_ref, s_ref, h_ref, o_ref):
    acc = (x_ref[...] + s_ref[...] * (r1_ref[...] + r2_ref[...])) * (1.0 / 3.0)
    t = _lrelu(jnp.dot(acc, h_ref[...], preferred_element_type=jnp.float32))
    t2 = _lrelu(lax.dot_general(t, h_ref[...], (((1,), (1,)), ((), ())),
                                preferred_element_type=jnp.float32))
    o_ref[...] = acc + t2


def _fuse(x, r1, r2, s, h, out_rows):
    n = x.shape[0]
    blk = 2000
    grid = n // blk
    return pl.pallas_call(
        _fuse_body,
        grid=(grid,),
        in_specs=[pl.BlockSpec((blk, D), lambda i: (i, 0)),
                  pl.BlockSpec((blk, D), lambda i: (i, 0)),
                  pl.BlockSpec((blk, D), lambda i: (i, 0)),
                  pl.BlockSpec((blk, 1), lambda i: (i, 0)),
                  pl.BlockSpec((D, H), lambda i: (0, 0))],
        out_specs=pl.BlockSpec((blk, D), lambda i: (i, 0)),
        out_shape=jax.ShapeDtypeStruct((out_rows, D), jnp.float32),
    )(x, r1, r2, s[:, None], h)


def _score_body(u_ref, b_ref, o_ref):
    u = u_ref[...]
    br = b_ref[...]
    p0 = jnp.sum(u * br[:, :D], axis=1, keepdims=True)
    p1 = jnp.sum(u * br[:, D:], axis=1, keepdims=True)
    o_ref[...] = jnp.concatenate([p0, p1], axis=1)


def _pad_scatter(idx, e_pad):
    pad = jnp.full((e_pad - idx.shape[0],), 100_000_000, jnp.int32)
    return jnp.concatenate([idx.astype(jnp.int32), pad]).reshape(-1, CHUNK)


def _pad_gather(idx, e_pad, n):
    pad = jnp.arange(e_pad - idx.shape[0], dtype=jnp.int32) % n
    return jnp.concatenate([idx.astype(jnp.int32), pad]).reshape(-1, CHUNK)


_spmm_a = _make_spmm(2 * R50, EP8, R50, A50, 2)
_spmm_b = _make_spmm(2 * R20, EP8, R50, A50, 2)
_spmm_c = _make_spmm(2 * R50, EP8, R20, A20, 8)
_spmm_d = _make_spmm(2 * R50, EPB, R20, A20, 8)


def kernel(users_feature, items_feature, bundles_feature, IL_user_hyper, IL_item_hyper, BL_user_hyper, BL_bundle_hyper, ui_src, ui_dst, ub_src, ub_dst, bi_src, bi_dst, users, bundles):
    ui_s_sc = _pad_scatter(ui_src, EP8)
    ui_s_ga = _pad_gather(ui_src, EP8, NU)
    ui_d_sc = _pad_scatter(ui_dst, EP8)
    ui_d_ga = _pad_gather(ui_dst, EP8, NI)
    ub_s_sc = _pad_scatter(ub_src, EP8)
    ub_s_ga = _pad_gather(ub_src, EP8, NU)
    ub_d_sc = _pad_scatter(ub_dst, EP8)
    ub_d_ga = _pad_gather(ub_dst, EP8, NB)
    bi_s_sc = _pad_scatter(bi_src, EPB)
    bi_d_ga = _pad_gather(bi_dst, EPB, NI)

    deg_ui_u, deg_ui_i, deg_ub_u, deg_ub_b, deg_bi = _hist5(
        ui_s_sc, ui_d_sc, ub_s_sc, ub_d_sc, bi_s_sc)

    ra_ui = _ew(_dinv_body, deg_ui_u)
    rb_ui = _ew(_dinv_body, deg_ui_i)
    ra_ub = _ew(_dinv_body, deg_ub_u)
    rb_ub = _ew(_dinv_body, deg_ub_b)
    inv_bi = _ew(_invdeg_body, deg_bi)

    yu_ui = _rowscale(users_feature, ra_ui[:NU], 2 * R50)
    yi_ui = _rowscale(items_feature, rb_ui[:NI], 2 * R50)
    yu_ub = _rowscale(users_feature, ra_ub[:NU], 2 * R50)
    yb_ub = _rowscale(bundles_feature, rb_ub[:NB], 2 * R20)

    raw_u1_il = _spmm_a(yi_ui, ui_d_ga, ui_s_sc)
    raw_i1 = _spmm_a(yu_ui, ui_s_ga, ui_d_sc)
    raw_u1_bl = _spmm_b(yb_ub, ub_d_ga, ub_s_sc)
    raw_b1 = _spmm_c(yu_ub, ub_s_ga, ub_d_sc)

    y2_i = _rowscale(raw_i1, rb_ui, 2 * R50, square=True)
    y2_u_il = _rowscale(raw_u1_il, ra_ui, 2 * R50, square=True)
    y2_b = _rowscale(raw_b1, rb_ub, 2 * R20, square=True)
    y2_u_bl = _rowscale(raw_u1_bl, ra_ub, 2 * R50, square=True)

    raw_u2_il = _spmm_a(y2_i, ui_d_ga, ui_s_sc)
    raw_i2 = _spmm_a(y2_u_il, ui_s_ga, ui_d_sc)
    raw_u2_bl = _spmm_b(y2_b, ub_d_ga, ub_s_sc)
    raw_b2 = _spmm_c(y2_u_bl, ub_s_ga, ub_d_sc)

    IL_users_f = _fuse(users_feature, raw_u1_il[:NU], raw_u2_il[:NU], ra_ui[:NU], IL_user_hyper, NU)
    IL_items_f = _fuse(items_feature, raw_i1[:NI], raw_i2[:NI], rb_ui[:NI], IL_item_hyper, 2 * R50)
    BL_users_f = _fuse(users_feature, raw_u1_bl[:NU], raw_u2_bl[:NU], ra_ub[:NU], BL_user_hyper, NU)
    BL_bundles_f = _fuse(bundles_feature, raw_b1[:NB], raw_b2[:NB], rb_ub[:NB], BL_bundle_hyper, NB)

    raw_bi = _spmm_d(IL_items_f, bi_d_ga, bi_s_sc)
    il_bundles = _rowscale(raw_bi, inv_bi, 2 * R20)

    u_rows, b_rows = _batch_gather(
        IL_users_f, BL_users_f, users.astype(jnp.int32),
        il_bundles, BL_bundles_f, bundles.reshape(-1).astype(jnp.int32))

    pred = pl.pallas_call(
        _score_body,
        out_shape=jax.ShapeDtypeStruct((4096, 2), jnp.float32),
    )(u_rows, b_rows.reshape(4096, 2 * D))
    return pred


# speedup vs baseline: 24.2934x; 1.3307x over previous
import functools

import jax
import jax.numpy as jnp
from jax import lax
from jax.experimental import pallas as pl
from jax.experimental.pallas import tpu as pltpu
from jax.experimental.pallas import tpu_sc as plsc

NU, NI, NB, D, H = 50000, 50000, 20000, 64, 128
LANES = 16
CHUNK = 128
NTILE = 16

R50, A50 = 25600, 26624
R20, A20 = 10240, 12288
NBUF = 8
EP8 = 819200
EPB = 655360

_mesh = plsc.VectorSubcoreMesh(core_axis_name="c", subcore_axis_name="s")


def _make_spmm(n_gather, e_pad, r, acc_rows, nbuf):
    ept = e_pad // NTILE
    n_ch = ept // CHUNK
    rpt = r // NTILE
    zpt = acc_rows // NTILE // CHUNK
    dmask = acc_rows - r - 1

    n_body = n_ch // nbuf
    half = nbuf // 2

    @functools.partial(
        pl.kernel,
        out_type=jax.ShapeDtypeStruct((2 * r, D), jnp.float32),
        mesh=_mesh,
        compiler_params=pltpu.CompilerParams(use_tc_tiling_on_sc=False),
        scratch_types=[
            pltpu.VMEM_SHARED((acc_rows, D), jnp.float32),
            pltpu.VMEM((nbuf, CHUNK), jnp.int32),
            pltpu.VMEM((nbuf, CHUNK), jnp.int32),
            pltpu.VMEM((nbuf, CHUNK), jnp.int32),
            pltpu.VMEM((nbuf, CHUNK, D), jnp.float32),
            pltpu.SemaphoreType.DMA,
            pltpu.SemaphoreType.DMA,
        ],
    )
    def spmm(y_hbm, gidx_hbm, didx_hbm, out_hbm, acc, gblk, dblk, sblk, rbuf, gsem, ssem):
        c = lax.axis_index("c")
        s = lax.axis_index("s")
        base = c * r
        iota = lax.iota(jnp.int32, LANES)

        def zrow(i, carry):
            for k in range(D // LANES):
                rbuf[0, i, pl.ds(k * LANES, LANES)] = jnp.zeros((LANES,), jnp.float32)
            return carry

        lax.fori_loop(0, CHUNK, zrow, 0)
        for z in range(zpt):
            pltpu.sync_copy(rbuf.at[0],
                            acc.at[pl.ds(s * (acc_rows // NTILE) + z * CHUNK, CHUNK)])
        plsc.subcore_barrier()

        rbase = s * n_ch

        def body(g, carry):
            row0 = rbase + g * nbuf
            d1 = pltpu.async_copy(gidx_hbm.at[pl.ds(row0, nbuf)], gblk, gsem)
            d2 = pltpu.async_copy(didx_hbm.at[pl.ds(row0, nbuf)], dblk, gsem)
            d1.wait()
            d2.wait()
            gd = [pltpu.async_copy(y_hbm.at[gblk.at[b]], rbuf.at[b], gsem)
                  for b in range(nbuf)]
            for b in range(nbuf):
                for j in range(CHUNK // LANES):
                    dv = dblk[b, pl.ds(j * LANES, LANES)]
                    loc = dv - base
                    ok = (loc >= 0) & (loc < r)
                    spread = (iota + ((g * nbuf + b) * CHUNK + j * LANES)) & dmask
                    sblk[b, pl.ds(j * LANES, LANES)] = jnp.where(ok, loc, r + spread)
            sd = []
            for b in range(half):
                gd[b].wait()
                sd.append(pltpu.async_copy(rbuf.at[b], acc.at[sblk.at[b]], ssem, add=True))
            for b in range(half, nbuf):
                gd[b].wait()
                sd.append(pltpu.async_copy(rbuf.at[b], acc.at[sblk.at[b]], ssem, add=True))
            for d in sd:
                d.wait()
            return carry

        lax.fori_loop(0, n_body, body, 0)
        plsc.subcore_barrier()
        pltpu.sync_copy(acc.at[pl.ds(s * rpt, rpt)], out_hbm.at[pl.ds(base + s * rpt, rpt)])

    return spmm


_SENT = 100_000_000
_BIN_CFG = [(EP8, R50), (EP8, R50), (EP8, R50), (EP8, R20), (EPB, R20)]


def _bin_rc(e_pad):
    return e_pad // 32 // CHUNK + 2


_BIN_OUT = []
for _ep, _r in _BIN_CFG:
    _BIN_OUT += [jax.ShapeDtypeStruct((64 * _bin_rc(_ep) * CHUNK,), jnp.int32),
                 jax.ShapeDtypeStruct((64 * _bin_rc(_ep) * CHUNK,), jnp.int32),
                 jax.ShapeDtypeStruct((512,), jnp.int32)]


@functools.partial(
    pl.kernel,
    out_type=tuple(_BIN_OUT),
    mesh=_mesh,
    compiler_params=pltpu.CompilerParams(use_tc_tiling_on_sc=False,
                                         needs_layout_passes=False),
    scratch_types=[
        pltpu.VMEM((8 * CHUNK,), jnp.int32),
        pltpu.VMEM((8 * CHUNK,), jnp.int32),
        pltpu.VMEM((272,), jnp.int32),
        pltpu.VMEM((272,), jnp.int32),
        pltpu.VMEM((272,), jnp.int32),
        pltpu.VMEM((272,), jnp.int32),
        pltpu.VMEM((CHUNK,), jnp.int32),
        pltpu.VMEM((CHUNK,), jnp.int32),
        pltpu.VMEM((LANES,), jnp.int32),
        pltpu.SemaphoreType.DMA,
    ],
)
def _bin5(g1, d1, g2, d2, g3, d3, g4, d4, g5, d5,
          bg1, bs1, c1, bg2, bs2, c2, bg3, bs3, c3, bg4, bs4, c4, bg5, bs5, c5,
          gblk, dblk, bg0, bs0, bgh1, bsh1, padg, pads, cbuf, sem):
    c = lax.axis_index("c")
    s = lax.axis_index("s")
    w = s * 2 + c
    iota = lax.iota(jnp.int32, LANES)
    for j in range(CHUNK // LANES):
        padg[pl.ds(j * LANES, LANES)] = (iota + j * LANES) & 127
        pads[pl.ds(j * LANES, LANES)] = jnp.full((LANES,), _SENT, jnp.int32)

    for (gidx, didx, bgo, bso, co, (e_pad, r)) in (
            (g1, d1, bg1, bs1, c1, _BIN_CFG[0]),
            (g2, d2, bg2, bs2, c2, _BIN_CFG[1]),
            (g3, d3, bg3, bs3, c3, _BIN_CFG[2]),
            (g4, d4, bg4, bs4, c4, _BIN_CFG[3]),
            (g5, d5, bg5, bs5, c5, _BIN_CFG[4])):
        rpp = e_pad // 32 // CHUNK
        rc = rpp + 2
        in_off = w * rpp * CHUNK

        def chunk_body(ch, carry, bgo=bgo, bso=bso, r=r, rc=rc):
            offs = [carry[0], carry[2]]
            fs = [carry[1], carry[3]]
            for h, (bg, bs) in enumerate(((bg0, bs0), (bgh1, bsh1))):
                lo = h * r
                for j in range(CHUNK // LANES):
                    d16 = dblk[pl.ds(ch * CHUNK + j * LANES, LANES)]
                    g16 = gblk[pl.ds(ch * CHUNK + j * LANES, LANES)]
                    m = (d16 >= lo) & (d16 < lo + r)
                    plsc.store_compressed(bg.at[pl.ds(offs[h], LANES)], g16, mask=m)
                    plsc.store_compressed(bs.at[pl.ds(offs[h], LANES)], d16, mask=m)
                    cum = plsc.cumsum(jnp.where(m, 1, 0).astype(jnp.int32))
                    offs[h] = offs[h] + cum[15]
                do = offs[h] >= CHUNK
                orow = (h * 32 + w) * rc + fs[h]

                @pl.when(do)
                def _(bg=bg, bs=bs, orow=orow, bgo=bgo, bso=bso):
                    pltpu.sync_copy(bg.at[pl.ds(0, CHUNK)], bgo.at[pl.ds(orow * CHUNK, CHUNK)])
                    pltpu.sync_copy(bs.at[pl.ds(0, CHUNK)], bso.at[pl.ds(orow * CHUNK, CHUNK)])
                    for grp in range(CHUNK // LANES):
                        sl = pl.ds(grp * LANES, LANES)
                        sh = pl.ds(CHUNK + grp * LANES, LANES)
                        bg[sl] = bg[sh]
                        bs[sl] = bs[sh]

                offs[h] = jnp.where(do, offs[h] - CHUNK, offs[h])
                fs[h] = jnp.where(do, fs[h] + 1, fs[h])
            return offs[0], fs[0], offs[1], fs[1]

        def super_body(gsup, carry):
            off0 = in_off + gsup * 8 * CHUNK
            dd1 = pltpu.async_copy(gidx.at[pl.ds(off0, 8 * CHUNK)], gblk, sem)
            dd2 = pltpu.async_copy(didx.at[pl.ds(off0, 8 * CHUNK)], dblk, sem)
            dd1.wait()
            dd2.wait()
            return lax.fori_loop(0, 8, chunk_body, carry)

        off0, f0, off1, f1 = lax.fori_loop(0, rpp // 8, super_body,
                                           (jnp.int32(0),) * 4)

        nchs = []
        for h, (bg, bs), off, f in ((0, (bg0, bs0), off0, f0),
                                    (1, (bgh1, bsh1), off1, f1)):
            for j in range(CHUNK // LANES):
                bg[pl.ds(off + j * LANES, LANES)] = (iota + j * LANES) & 127
                bs[pl.ds(off + j * LANES, LANES)] = jnp.full((LANES,), _SENT, jnp.int32)
            t1 = f + (off > 0).astype(jnp.int32)
            orow = (h * 32 + w) * rc

            @pl.when(off > 0)
            def _(bg=bg, bs=bs, orow=orow, f=f, bgo=bgo, bso=bso):
                pltpu.sync_copy(bg.at[pl.ds(0, CHUNK)], bgo.at[pl.ds((orow + f) * CHUNK, CHUNK)])
                pltpu.sync_copy(bs.at[pl.ds(0, CHUNK)], bso.at[pl.ds((orow + f) * CHUNK, CHUNK)])

            @pl.when(t1 % 2 == 1)
            def _(orow=orow, t1=t1, bgo=bgo, bso=bso):
                pltpu.sync_copy(padg, bgo.at[pl.ds((orow + t1) * CHUNK, CHUNK)])
                pltpu.sync_copy(pads, bso.at[pl.ds((orow + t1) * CHUNK, CHUNK)])

            nchs.append(t1 + t1 % 2)

        cvec = jnp.where(iota == 0, nchs[0],
                         jnp.where(iota == 1, nchs[1], 0)).astype(jnp.int32)
        cbuf[...] = cvec
        pltpu.sync_copy(cbuf, co.at[pl.ds(w * LANES, LANES)])


def _make_spmm_bin(n_gather, e_pad, r, acc_rows):
    rc = _bin_rc(e_pad)
    rpt = r // NTILE
    zpt = acc_rows // NTILE // CHUNK
    dmask = acc_rows - r - 1

    @functools.partial(
        pl.kernel,
        out_type=jax.ShapeDtypeStruct((2 * r, D), jnp.float32),
        mesh=_mesh,
        compiler_params=pltpu.CompilerParams(use_tc_tiling_on_sc=False),
        scratch_types=[
            pltpu.VMEM_SHARED((acc_rows, D), jnp.float32),
            pltpu.VMEM((2 * CHUNK,), jnp.int32),
            pltpu.VMEM((2 * CHUNK,), jnp.int32),
            pltpu.VMEM((2, CHUNK), jnp.int32),
            pltpu.VMEM((2, CHUNK, D), jnp.float32),
            pltpu.VMEM((2 * LANES,), jnp.int32),
            pltpu.SemaphoreType.DMA,
            pltpu.SemaphoreType.DMA,
        ],
    )
    def spmm(y_hbm, bg_hbm, bs_hbm, cnt_hbm, out_hbm,
             acc, gblk, dblk, sblk, rbuf, cvbuf, gsem, ssem):
        c = lax.axis_index("c")
        s = lax.axis_index("s")
        base = c * r
        iota = lax.iota(jnp.int32, LANES)

        def zrow(i, carry):
            for k in range(D // LANES):
                rbuf[0, i, pl.ds(k * LANES, LANES)] = jnp.zeros((LANES,), jnp.float32)
            return carry

        lax.fori_loop(0, CHUNK, zrow, 0)
        for z in range(zpt):
            pltpu.sync_copy(rbuf.at[0],
                            acc.at[pl.ds(s * (acc_rows // NTILE) + z * CHUNK, CHUNK)])
        plsc.subcore_barrier()

        pltpu.sync_copy(cnt_hbm.at[pl.ds(s * 2 * LANES, 2 * LANES)], cvbuf)
        v0 = cvbuf[pl.ds(0, LANES)]
        v1 = cvbuf[pl.ds(LANES, LANES)]
        n0 = jnp.where(c == 0, v0[0], v0[1])
        n1 = jnp.where(c == 0, v1[0], v1[1])

        for t, nch in ((0, n0), (1, n1)):
            reg = (c * 32 + (2 * s + t)) * rc

            def body(g, carry):
                off0 = (reg + g * 2) * CHUNK
                dd1 = pltpu.async_copy(bg_hbm.at[pl.ds(off0, 2 * CHUNK)], gblk, gsem)
                dd2 = pltpu.async_copy(bs_hbm.at[pl.ds(off0, 2 * CHUNK)], dblk, gsem)
                dd1.wait()
                dd2.wait()
                gd = [pltpu.async_copy(y_hbm.at[gblk.at[pl.ds(b * CHUNK, CHUNK)]],
                                       rbuf.at[b], gsem) for b in range(2)]
                for b in range(2):
                    for j in range(CHUNK // LANES):
                        dv = dblk[pl.ds(b * CHUNK + j * LANES, LANES)]
                        loc = dv - base
                        ok = (loc >= 0) & (loc < r)
                        spread = (iota + ((g * 2 + b) * CHUNK + j * LANES)) & dmask
                        sblk[b, pl.ds(j * LANES, LANES)] = jnp.where(ok, loc, r + spread)
                sd = []
                for b in range(2):
                    gd[b].wait()
                    sd.append(pltpu.async_copy(rbuf.at[b], acc.at[sblk.at[b]],
                                               ssem, add=True))
                for d in sd:
                    d.wait()
                return carry

            lax.fori_loop(0, nch // 2, body, 0)

        plsc.subcore_barrier()
        pltpu.sync_copy(acc.at[pl.ds(s * rpt, rpt)], out_hbm.at[pl.ds(base + s * rpt, rpt)])

    return spmm


_HIST_CFG = [(EP8, R50, A50), (EP8, R50, A50), (EP8, R50, A50),
             (EP8, R20, A20), (EPB, R20, A20)]


@functools.partial(
    pl.kernel,
    out_type=tuple(jax.ShapeDtypeStruct((2 * r,), jnp.float32) for (_, r, _) in _HIST_CFG),
    mesh=_mesh,
    compiler_params=pltpu.CompilerParams(use_tc_tiling_on_sc=False),
    scratch_types=[
        pltpu.VMEM_SHARED((A50,), jnp.float32),
        pltpu.VMEM((NBUF, CHUNK), jnp.int32),
        pltpu.VMEM((NBUF, CHUNK), jnp.int32),
        pltpu.VMEM((CHUNK,), jnp.float32),
        pltpu.VMEM((CHUNK,), jnp.float32),
        pltpu.SemaphoreType.DMA,
        pltpu.SemaphoreType.DMA,
    ],
)
def _hist5(i1, i2, i3, i4, i5, o1, o2, o3, o4, o5, acc, dblk, sblk, ones, zbuf, gsem, ssem):
    c = lax.axis_index("c")
    s = lax.axis_index("s")
    iota = lax.iota(jnp.int32, LANES)
    for j in range(CHUNK // LANES):
        ones[pl.ds(j * LANES, LANES)] = jnp.full((LANES,), 1.0, jnp.float32)
        zbuf[pl.ds(j * LANES, LANES)] = jnp.zeros((LANES,), jnp.float32)

    for idx_hbm, out_hbm, (e_pad, r, acc_rows) in zip(
            (i1, i2, i3, i4, i5), (o1, o2, o3, o4, o5), _HIST_CFG):
        base = c * r
        dmask = acc_rows - r - 1
        n_ch = e_pad // NTILE // CHUNK
        rpt = r // NTILE
        for z in range(acc_rows // NTILE // CHUNK):
            pltpu.sync_copy(zbuf, acc.at[pl.ds(s * (acc_rows // NTILE) + z * CHUNK, CHUNK)])
        plsc.subcore_barrier()

        rbase = s * n_ch

        def body(g, carry):
            row0 = rbase + g * NBUF
            pltpu.async_copy(idx_hbm.at[pl.ds(row0, NBUF)], dblk, gsem).wait()
            for b in range(NBUF):
                for j in range(CHUNK // LANES):
                    dv = dblk[b, pl.ds(j * LANES, LANES)]
                    loc = dv - base
                    ok = (loc >= 0) & (loc < r)
                    spread = (iota + ((g * NBUF + b) * CHUNK + j * LANES)) & dmask
                    sblk[b, pl.ds(j * LANES, LANES)] = jnp.where(ok, loc, r + spread)
            sd = [pltpu.async_copy(ones, acc.at[sblk.at[b]], ssem, add=True)
                  for b in range(NBUF)]
            for d in sd:
                d.wait()
            return carry

        lax.fori_loop(0, n_ch // NBUF, body, 0)
        plsc.subcore_barrier()
        pltpu.sync_copy(acc.at[pl.ds(s * rpt, rpt)], out_hbm.at[pl.ds(base + s * rpt, rpt)])
        plsc.subcore_barrier()


@functools.partial(
    pl.kernel,
    out_type=(jax.ShapeDtypeStruct((4096, D), jnp.float32),
              jax.ShapeDtypeStruct((8192, D), jnp.float32)),
    mesh=_mesh,
    compiler_params=pltpu.CompilerParams(use_tc_tiling_on_sc=False),
    scratch_types=[
        pltpu.VMEM((CHUNK,), jnp.int32),
        pltpu.VMEM((CHUNK, D), jnp.float32),
        pltpu.VMEM((CHUNK, D), jnp.float32),
        pltpu.SemaphoreType.DMA,
    ],
)
def _batch_gather(ilu, blu, uidx, ilb, blb, bidx, u_out, b_out,
                  ibuf, r1, r2, sem):
    c = lax.axis_index("c")
    s = lax.axis_index("s")
    wid = s * 2 + c

    def addrows(i, carry):
        for k in range(D // LANES):
            sl = pl.ds(k * LANES, LANES)
            r1[i, sl] = r1[i, sl] + r2[i, sl]
        return carry

    off = wid * CHUNK
    pltpu.sync_copy(uidx.at[pl.ds(off, CHUNK)], ibuf)
    pltpu.async_copy(ilu.at[ibuf], r1, sem).wait()
    pltpu.async_copy(blu.at[ibuf], r2, sem).wait()
    lax.fori_loop(0, CHUNK, addrows, 0)
    pltpu.sync_copy(r1, u_out.at[pl.ds(off, CHUNK)])

    for cc in range(2):
        off = wid * 2 * CHUNK + cc * CHUNK
        pltpu.sync_copy(bidx.at[pl.ds(off, CHUNK)], ibuf)
        pltpu.async_copy(ilb.at[ibuf], r1, sem).wait()
        pltpu.async_copy(blb.at[ibuf], r2, sem).wait()
        lax.fori_loop(0, CHUNK, addrows, 0)
        pltpu.sync_copy(r1, b_out.at[pl.ds(off, CHUNK)])


def _dinv_body(x_ref, o_ref):
    o_ref[...] = 1.0 / (jnp.sqrt(x_ref[...]) + 1e-8)


def _invdeg_body(x_ref, o_ref):
    o_ref[...] = 1.0 / (x_ref[...] + 1e-8)


def _ew(body, x):
    n = x.shape[0]
    x2 = x.reshape(n // 128, 128)
    out = pl.pallas_call(body, out_shape=jax.ShapeDtypeStruct(x2.shape, jnp.float32))(x2)
    return out.reshape(n)


def _rowscale_body(x_ref, s_ref, o_ref, *, square):
    sc = s_ref[...]
    if square:
        sc = sc * sc
    o_ref[...] = x_ref[...] * sc


def _rowscale(x, s, out_rows, square=False):
    n = x.shape[0]
    blk = 2000 if n % 2000 == 0 else 2048
    grid = n // blk
    return pl.pallas_call(
        functools.partial(_rowscale_body, square=square),
        grid=(grid,),
        in_specs=[pl.BlockSpec((blk, D), lambda i: (i, 0)),
                  pl.BlockSpec((blk, 1), lambda i: (i, 0))],
        out_specs=pl.BlockSpec((blk, D), lambda i: (i, 0)),
        out_shape=jax.ShapeDtypeStruct((out_rows, D), jnp.float32),
    )(x, s[:, None])


def _lrelu(x):
    return jnp.where(x > 0, x, 0.05 * x)


def _fuse_body(x_ref, r1_ref, r2_ref, s_ref, h_ref, o_ref):
    acc = (x_ref[...] + s_ref[...] * (r1_ref[...] + r2_ref[...])) * (1.0 / 3.0)
    t = _lrelu(jnp.dot(acc, h_ref[...], preferred_element_type=jnp.float32))
    t2 = _lrelu(lax.dot_general(t, h_ref[...], (((1,), (1,)), ((), ())),
                                preferred_element_type=jnp.float32))
    o_ref[...] = acc + t2


def _fuse(x, r1, r2, s, h, out_rows):
    n = x.shape[0]
    blk = 2000
    grid = n // blk
    return pl.pallas_call(
        _fuse_body,
        grid=(grid,),
        in_specs=[pl.BlockSpec((blk, D), lambda i: (i, 0)),
                  pl.BlockSpec((blk, D), lambda i: (i, 0)),
                  pl.BlockSpec((blk, D), lambda i: (i, 0)),
                  pl.BlockSpec((blk, 1), lambda i: (i, 0)),
                  pl.BlockSpec((D, H), lambda i: (0, 0))],
        out_specs=pl.BlockSpec((blk, D), lambda i: (i, 0)),
        out_shape=jax.ShapeDtypeStruct((out_rows, D), jnp.float32),
    )(x, r1, r2, s[:, None], h)


def _score_body(u_ref, b_ref, o_ref):
    u = u_ref[...]
    br = b_ref[...]
    p0 = jnp.sum(u * br[:, :D], axis=1, keepdims=True)
    p1 = jnp.sum(u * br[:, D:], axis=1, keepdims=True)
    o_ref[...] = jnp.concatenate([p0, p1], axis=1)


def _pad_scatter(idx, e_pad):
    pad = jnp.full((e_pad - idx.shape[0],), 100_000_000, jnp.int32)
    return jnp.concatenate([idx.astype(jnp.int32), pad]).reshape(-1, CHUNK)


def _pad_gather(idx, e_pad, n):
    pad = jnp.arange(e_pad - idx.shape[0], dtype=jnp.int32) % n
    return jnp.concatenate([idx.astype(jnp.int32), pad]).reshape(-1, CHUNK)


_spmm_a = _make_spmm_bin(2 * R50, EP8, R50, A50)
_spmm_b = _make_spmm_bin(2 * R20, EP8, R50, A50)
_spmm_c = _make_spmm_bin(2 * R50, EP8, R20, A20)
_spmm_d = _make_spmm_bin(2 * R50, EPB, R20, A20)


def kernel(users_feature, items_feature, bundles_feature, IL_user_hyper, IL_item_hyper, BL_user_hyper, BL_bundle_hyper, ui_src, ui_dst, ub_src, ub_dst, bi_src, bi_dst, users, bundles):
    ui_s_sc = _pad_scatter(ui_src, EP8)
    ui_s_ga = _pad_gather(ui_src, EP8, NU)
    ui_d_sc = _pad_scatter(ui_dst, EP8)
    ui_d_ga = _pad_gather(ui_dst, EP8, NI)
    ub_s_sc = _pad_scatter(ub_src, EP8)
    ub_s_ga = _pad_gather(ub_src, EP8, NU)
    ub_d_sc = _pad_scatter(ub_dst, EP8)
    ub_d_ga = _pad_gather(ub_dst, EP8, NB)
    bi_s_sc = _pad_scatter(bi_src, EPB)
    bi_d_ga = _pad_gather(bi_dst, EPB, NI)

    deg_ui_u, deg_ui_i, deg_ub_u, deg_ub_b, deg_bi = _hist5(
        ui_s_sc, ui_d_sc, ub_s_sc, ub_d_sc, bi_s_sc)

    (bgA, bsA, cA, bgB, bsB, cB, bgC, bsC, cC,
     bgD, bsD, cD, bgE, bsE, cE) = _bin5(
        ui_d_ga.reshape(-1), ui_s_sc.reshape(-1),
        ui_s_ga.reshape(-1), ui_d_sc.reshape(-1),
        ub_d_ga.reshape(-1), ub_s_sc.reshape(-1),
        ub_s_ga.reshape(-1), ub_d_sc.reshape(-1),
        bi_d_ga.reshape(-1), bi_s_sc.reshape(-1))

    ra_ui = _ew(_dinv_body, deg_ui_u)
    rb_ui = _ew(_dinv_body, deg_ui_i)
    ra_ub = _ew(_dinv_body, deg_ub_u)
    rb_ub = _ew(_dinv_body, deg_ub_b)
    inv_bi = _ew(_invdeg_body, deg_bi)

    yu_ui = _rowscale(users_feature, ra_ui[:NU], 2 * R50)
    yi_ui = _rowscale(items_feature, rb_ui[:NI], 2 * R50)
    yu_ub = _rowscale(users_feature, ra_ub[:NU], 2 * R50)
    yb_ub = _rowscale(bundles_feature, rb_ub[:NB], 2 * R20)

    raw_u1_il = _spmm_a(yi_ui, bgA, bsA, cA)
    raw_i1 = _spmm_a(yu_ui, bgB, bsB, cB)
    raw_u1_bl = _spmm_b(yb_ub, bgC, bsC, cC)
    raw_b1 = _spmm_c(yu_ub, bgD, bsD, cD)

    y2_i = _rowscale(raw_i1, rb_ui, 2 * R50, square=True)
    y2_u_il = _rowscale(raw_u1_il, ra_ui, 2 * R50, square=True)
    y2_b = _rowscale(raw_b1, rb_ub, 2 * R20, square=True)
    y2_u_bl = _rowscale(raw_u1_bl, ra_ub, 2 * R50, square=True)

    raw_u2_il = _spmm_a(y2_i, bgA, bsA, cA)
    raw_i2 = _spmm_a(y2_u_il, bgB, bsB, cB)
    raw_u2_bl = _spmm_b(y2_b, bgC, bsC, cC)
    raw_b2 = _spmm_c(y2_u_bl, bgD, bsD, cD)

    IL_users_f = _fuse(users_feature, raw_u1_il[:NU], raw_u2_il[:NU], ra_ui[:NU], IL_user_hyper, NU)
    IL_items_f = _fuse(items_feature, raw_i1[:NI], raw_i2[:NI], rb_ui[:NI], IL_item_hyper, 2 * R50)
    BL_users_f = _fuse(users_feature, raw_u1_bl[:NU], raw_u2_bl[:NU], ra_ub[:NU], BL_user_hyper, NU)
    BL_bundles_f = _fuse(bundles_feature, raw_b1[:NB], raw_b2[:NB], rb_ub[:NB], BL_bundle_hyper, NB)

    raw_bi = _spmm_d(IL_items_f, bgE, bsE, cE)
    il_bundles = _rowscale(raw_bi, inv_bi, 2 * R20)

    u_rows, b_rows = _batch_gather(
        IL_users_f, BL_users_f, users.astype(jnp.int32),
        il_bundles, BL_bundles_f, bundles.reshape(-1).astype(jnp.int32))

    pred = pl.pallas_call(
        _score_body,
        out_shape=jax.ShapeDtypeStruct((4096, 2), jnp.float32),
    )(u_rows, b_rows.reshape(4096, 2 * D))
    return pred


# speedup vs baseline: 26.9859x; 1.1108x over previous
import functools

import jax
import jax.numpy as jnp
from jax import lax
from jax.experimental import pallas as pl
from jax.experimental.pallas import tpu as pltpu
from jax.experimental.pallas import tpu_sc as plsc

NU, NI, NB, D, H = 50000, 50000, 20000, 64, 128
LANES = 16
CHUNK = 128
NTILE = 16

R50, A50 = 25600, 26624
R20, A20 = 10240, 12288
NBUF = 8
EP8 = 819200
EPB = 655360

_mesh = plsc.VectorSubcoreMesh(core_axis_name="c", subcore_axis_name="s")


def _make_spmm(n_gather, e_pad, r, acc_rows, nbuf):
    ept = e_pad // NTILE
    n_ch = ept // CHUNK
    rpt = r // NTILE
    zpt = acc_rows // NTILE // CHUNK
    dmask = acc_rows - r - 1

    n_body = n_ch // nbuf
    half = nbuf // 2

    @functools.partial(
        pl.kernel,
        out_type=jax.ShapeDtypeStruct((2 * r, D), jnp.float32),
        mesh=_mesh,
        compiler_params=pltpu.CompilerParams(use_tc_tiling_on_sc=False),
        scratch_types=[
            pltpu.VMEM_SHARED((acc_rows, D), jnp.float32),
            pltpu.VMEM((nbuf, CHUNK), jnp.int32),
            pltpu.VMEM((nbuf, CHUNK), jnp.int32),
            pltpu.VMEM((nbuf, CHUNK), jnp.int32),
            pltpu.VMEM((nbuf, CHUNK, D), jnp.float32),
            pltpu.SemaphoreType.DMA,
            pltpu.SemaphoreType.DMA,
        ],
    )
    def spmm(y_hbm, gidx_hbm, didx_hbm, out_hbm, acc, gblk, dblk, sblk, rbuf, gsem, ssem):
        c = lax.axis_index("c")
        s = lax.axis_index("s")
        base = c * r
        iota = lax.iota(jnp.int32, LANES)

        def zrow(i, carry):
            for k in range(D // LANES):
                rbuf[0, i, pl.ds(k * LANES, LANES)] = jnp.zeros((LANES,), jnp.float32)
            return carry

        lax.fori_loop(0, CHUNK, zrow, 0)
        for z in range(zpt):
            pltpu.sync_copy(rbuf.at[0],
                            acc.at[pl.ds(s * (acc_rows // NTILE) + z * CHUNK, CHUNK)])
        plsc.subcore_barrier()

        rbase = s * n_ch

        def body(g, carry):
            row0 = rbase + g * nbuf
            d1 = pltpu.async_copy(gidx_hbm.at[pl.ds(row0, nbuf)], gblk, gsem)
            d2 = pltpu.async_copy(didx_hbm.at[pl.ds(row0, nbuf)], dblk, gsem)
            d1.wait()
            d2.wait()
            gd = [pltpu.async_copy(y_hbm.at[gblk.at[b]], rbuf.at[b], gsem)
                  for b in range(nbuf)]
            for b in range(nbuf):
                for j in range(CHUNK // LANES):
                    dv = dblk[b, pl.ds(j * LANES, LANES)]
                    loc = dv - base
                    ok = (loc >= 0) & (loc < r)
                    spread = (iota + ((g * nbuf + b) * CHUNK + j * LANES)) & dmask
                    sblk[b, pl.ds(j * LANES, LANES)] = jnp.where(ok, loc, r + spread)
            sd = []
            for b in range(half):
                gd[b].wait()
                sd.append(pltpu.async_copy(rbuf.at[b], acc.at[sblk.at[b]], ssem, add=True))
            for b in range(half, nbuf):
                gd[b].wait()
                sd.append(pltpu.async_copy(rbuf.at[b], acc.at[sblk.at[b]], ssem, add=True))
            for d in sd:
                d.wait()
            return carry

        lax.fori_loop(0, n_body, body, 0)
        plsc.subcore_barrier()
        pltpu.sync_copy(acc.at[pl.ds(s * rpt, rpt)], out_hbm.at[pl.ds(base + s * rpt, rpt)])

    return spmm


_SENT = 100_000_000
_BIN_CFG = [(EP8, R50), (EP8, R50), (EP8, R50), (EP8, R20), (EPB, R20)]


def _bin_rc(e_pad):
    return e_pad // 32 // CHUNK + 2


_BIN_OUT = []
for _ep, _r in _BIN_CFG:
    _BIN_OUT += [jax.ShapeDtypeStruct((64 * _bin_rc(_ep) * CHUNK,), jnp.int32),
                 jax.ShapeDtypeStruct((64 * _bin_rc(_ep) * CHUNK,), jnp.int32),
                 jax.ShapeDtypeStruct((512,), jnp.int32)]


@functools.partial(
    pl.kernel,
    out_type=tuple(_BIN_OUT),
    mesh=_mesh,
    compiler_params=pltpu.CompilerParams(use_tc_tiling_on_sc=False,
                                         needs_layout_passes=False),
    scratch_types=[
        pltpu.VMEM((8 * CHUNK,), jnp.int32),
        pltpu.VMEM((8 * CHUNK,), jnp.int32),
        pltpu.VMEM((272,), jnp.int32),
        pltpu.VMEM((272,), jnp.int32),
        pltpu.VMEM((272,), jnp.int32),
        pltpu.VMEM((272,), jnp.int32),
        pltpu.VMEM((CHUNK,), jnp.int32),
        pltpu.VMEM((CHUNK,), jnp.int32),
        pltpu.VMEM((LANES,), jnp.int32),
        pltpu.SemaphoreType.DMA,
    ],
)
def _bin5(g1, d1, g2, d2, g3, d3, g4, d4, g5, d5,
          bg1, bs1, c1, bg2, bs2, c2, bg3, bs3, c3, bg4, bs4, c4, bg5, bs5, c5,
          gblk, dblk, bg0, bs0, bgh1, bsh1, padg, pads, cbuf, sem):
    c = lax.axis_index("c")
    s = lax.axis_index("s")
    w = s * 2 + c
    iota = lax.iota(jnp.int32, LANES)
    for j in range(CHUNK // LANES):
        padg[pl.ds(j * LANES, LANES)] = (iota + j * LANES) & 127
        pads[pl.ds(j * LANES, LANES)] = jnp.full((LANES,), _SENT, jnp.int32)

    for (gidx, didx, bgo, bso, co, (e_pad, r)) in (
            (g1, d1, bg1, bs1, c1, _BIN_CFG[0]),
            (g2, d2, bg2, bs2, c2, _BIN_CFG[1]),
            (g3, d3, bg3, bs3, c3, _BIN_CFG[2]),
            (g4, d4, bg4, bs4, c4, _BIN_CFG[3]),
            (g5, d5, bg5, bs5, c5, _BIN_CFG[4])):
        rpp = e_pad // 32 // CHUNK
        rc = rpp + 2
        in_off = w * rpp * CHUNK

        def chunk_body(ch, carry, bgo=bgo, bso=bso, r=r, rc=rc):
            offs = [carry[0], carry[2]]
            fs = [carry[1], carry[3]]
            for h, (bg, bs) in enumerate(((bg0, bs0), (bgh1, bsh1))):
                lo = h * r
                for j in range(CHUNK // LANES):
                    d16 = dblk[pl.ds(ch * CHUNK + j * LANES, LANES)]
                    g16 = gblk[pl.ds(ch * CHUNK + j * LANES, LANES)]
                    m = (d16 >= lo) & (d16 < lo + r)
                    plsc.store_compressed(bg.at[pl.ds(offs[h], LANES)], g16, mask=m)
                    plsc.store_compressed(bs.at[pl.ds(offs[h], LANES)], d16, mask=m)
                    cum = plsc.cumsum(jnp.where(m, 1, 0).astype(jnp.int32))
                    offs[h] = offs[h] + cum[15]
                do = offs[h] >= CHUNK
                orow = (h * 32 + w) * rc + fs[h]

                @pl.when(do)
                def _(bg=bg, bs=bs, orow=orow, bgo=bgo, bso=bso):
                    pltpu.sync_copy(bg.at[pl.ds(0, CHUNK)], bgo.at[pl.ds(orow * CHUNK, CHUNK)])
                    pltpu.sync_copy(bs.at[pl.ds(0, CHUNK)], bso.at[pl.ds(orow * CHUNK, CHUNK)])
                    for grp in range(CHUNK // LANES):
                        sl = pl.ds(grp * LANES, LANES)
                        sh = pl.ds(CHUNK + grp * LANES, LANES)
                        bg[sl] = bg[sh]
                        bs[sl] = bs[sh]

                offs[h] = jnp.where(do, offs[h] - CHUNK, offs[h])
                fs[h] = jnp.where(do, fs[h] + 1, fs[h])
            return offs[0], fs[0], offs[1], fs[1]

        def super_body(gsup, carry):
            off0 = in_off + gsup * 8 * CHUNK
            dd1 = pltpu.async_copy(gidx.at[pl.ds(off0, 8 * CHUNK)], gblk, sem)
            dd2 = pltpu.async_copy(didx.at[pl.ds(off0, 8 * CHUNK)], dblk, sem)
            dd1.wait()
            dd2.wait()
            return lax.fori_loop(0, 8, chunk_body, carry)

        off0, f0, off1, f1 = lax.fori_loop(0, rpp // 8, super_body,
                                           (jnp.int32(0),) * 4)

        nchs = []
        for h, (bg, bs), off, f in ((0, (bg0, bs0), off0, f0),
                                    (1, (bgh1, bsh1), off1, f1)):
            for j in range(CHUNK // LANES):
                bg[pl.ds(off + j * LANES, LANES)] = (iota + j * LANES) & 127
                bs[pl.ds(off + j * LANES, LANES)] = jnp.full((LANES,), _SENT, jnp.int32)
            t1 = f + (off > 0).astype(jnp.int32)
            orow = (h * 32 + w) * rc

            @pl.when(off > 0)
            def _(bg=bg, bs=bs, orow=orow, f=f, bgo=bgo, bso=bso):
                pltpu.sync_copy(bg.at[pl.ds(0, CHUNK)], bgo.at[pl.ds((orow + f) * CHUNK, CHUNK)])
                pltpu.sync_copy(bs.at[pl.ds(0, CHUNK)], bso.at[pl.ds((orow + f) * CHUNK, CHUNK)])

            @pl.when(t1 % 2 == 1)
            def _(orow=orow, t1=t1, bgo=bgo, bso=bso):
                pltpu.sync_copy(padg, bgo.at[pl.ds((orow + t1) * CHUNK, CHUNK)])
                pltpu.sync_copy(pads, bso.at[pl.ds((orow + t1) * CHUNK, CHUNK)])

            nchs.append(t1 + t1 % 2)

        cvec = jnp.where(iota == 0, nchs[0],
                         jnp.where(iota == 1, nchs[1], 0)).astype(jnp.int32)
        cbuf[...] = cvec
        pltpu.sync_copy(cbuf, co.at[pl.ds(w * LANES, LANES)])


def _make_spmm_bin(n_gather, e_pad, r, acc_rows):
    rc = _bin_rc(e_pad)
    rpt = r // NTILE
    zpt = acc_rows // NTILE // CHUNK
    dmask = acc_rows - r - 1

    @functools.partial(
        pl.kernel,
        out_type=jax.ShapeDtypeStruct((2 * r, D), jnp.float32),
        mesh=_mesh,
        compiler_params=pltpu.CompilerParams(use_tc_tiling_on_sc=False),
        scratch_types=[
            pltpu.VMEM_SHARED((acc_rows, D), jnp.float32),
            pltpu.VMEM((8 * CHUNK,), jnp.int32),
            pltpu.VMEM((8 * CHUNK,), jnp.int32),
            pltpu.VMEM((2, CHUNK), jnp.int32),
            pltpu.VMEM((2, CHUNK, D), jnp.float32),
            pltpu.VMEM((2 * LANES,), jnp.int32),
            pltpu.SemaphoreType.DMA,
            pltpu.SemaphoreType.DMA,
        ],
    )
    def spmm(y_hbm, bg_hbm, bs_hbm, cnt_hbm, out_hbm,
             acc, gblk, dblk, sblk, rbuf, cvbuf, gsem, ssem):
        c = lax.axis_index("c")
        s = lax.axis_index("s")
        base = c * r
        iota = lax.iota(jnp.int32, LANES)

        def zrow(i, carry):
            for k in range(D // LANES):
                rbuf[0, i, pl.ds(k * LANES, LANES)] = jnp.zeros((LANES,), jnp.float32)
            return carry

        lax.fori_loop(0, CHUNK, zrow, 0)
        for z in range(zpt):
            pltpu.sync_copy(rbuf.at[0],
                            acc.at[pl.ds(s * (acc_rows // NTILE) + z * CHUNK, CHUNK)])
        plsc.subcore_barrier()

        pltpu.sync_copy(cnt_hbm.at[pl.ds(s * 2 * LANES, 2 * LANES)], cvbuf)
        v0 = cvbuf[pl.ds(0, LANES)]
        v1 = cvbuf[pl.ds(LANES, LANES)]
        n0 = jnp.where(c == 0, v0[0], v0[1])
        n1 = jnp.where(c == 0, v1[0], v1[1])

        for t, nch in ((0, n0), (1, n1)):
            reg = (c * 32 + (2 * s + t)) * rc

            def pair(goff, ioff):
                gd = [pltpu.async_copy(
                    y_hbm.at[gblk.at[pl.ds((ioff + b) * CHUNK, CHUNK)]],
                    rbuf.at[b], gsem) for b in range(2)]
                for b in range(2):
                    for j in range(CHUNK // LANES):
                        dv = dblk[pl.ds((ioff + b) * CHUNK + j * LANES, LANES)]
                        loc = dv - base
                        ok = (loc >= 0) & (loc < r)
                        spread = (iota + ((goff + b) * CHUNK + j * LANES)) & dmask
                        sblk[b, pl.ds(j * LANES, LANES)] = jnp.where(ok, loc, r + spread)
                sd = []
                for b in range(2):
                    gd[b].wait()
                    sd.append(pltpu.async_copy(rbuf.at[b], acc.at[sblk.at[b]],
                                               ssem, add=True))
                for d in sd:
                    d.wait()

            def body8(g, carry):
                off0 = (reg + g * 8) * CHUNK
                dd1 = pltpu.async_copy(bg_hbm.at[pl.ds(off0, 8 * CHUNK)], gblk, gsem)
                dd2 = pltpu.async_copy(bs_hbm.at[pl.ds(off0, 8 * CHUNK)], dblk, gsem)
                dd1.wait()
                dd2.wait()
                for p in range(4):
                    pair(g * 8 + p * 2, p * 2)
                return carry

            def body2(g, carry):
                off0 = (reg + g * 2) * CHUNK
                dd1 = pltpu.async_copy(bg_hbm.at[pl.ds(off0, 2 * CHUNK)],
                                       gblk.at[pl.ds(0, 2 * CHUNK)], gsem)
                dd2 = pltpu.async_copy(bs_hbm.at[pl.ds(off0, 2 * CHUNK)],
                                       dblk.at[pl.ds(0, 2 * CHUNK)], gsem)
                dd1.wait()
                dd2.wait()
                pair(g * 2, 0)
                return carry

            n8 = nch // 8
            lax.fori_loop(0, n8, body8, 0)
            lax.fori_loop(n8 * 4, nch // 2, body2, 0)

        plsc.subcore_barrier()
        pltpu.sync_copy(acc.at[pl.ds(s * rpt, rpt)], out_hbm.at[pl.ds(base + s * rpt, rpt)])

    return spmm


_HIST_CFG = [(EP8, R50, A50), (EP8, R50, A50), (EP8, R50, A50),
             (EP8, R20, A20), (EPB, R20, A20)]


@functools.partial(
    pl.kernel,
    out_type=tuple(jax.ShapeDtypeStruct((2 * r,), jnp.float32) for (_, r, _) in _HIST_CFG),
    mesh=_mesh,
    compiler_params=pltpu.CompilerParams(use_tc_tiling_on_sc=False),
    scratch_types=[
        pltpu.VMEM_SHARED((A50,), jnp.float32),
        pltpu.VMEM((NBUF, CHUNK), jnp.int32),
        pltpu.VMEM((NBUF, CHUNK), jnp.int32),
        pltpu.VMEM((CHUNK,), jnp.float32),
        pltpu.VMEM((CHUNK,), jnp.float32),
        pltpu.SemaphoreType.DMA,
        pltpu.SemaphoreType.DMA,
    ],
)
def _hist5(i1, i2, i3, i4, i5, o1, o2, o3, o4, o5, acc, dblk, sblk, ones, zbuf, gsem, ssem):
    c = lax.axis_index("c")
    s = lax.axis_index("s")
    iota = lax.iota(jnp.int32, LANES)
    for j in range(CHUNK // LANES):
        ones[pl.ds(j * LANES, LANES)] = jnp.full((LANES,), 1.0, jnp.float32)
        zbuf[pl.ds(j * LANES, LANES)] = jnp.zeros((LANES,), jnp.float32)

    for idx_hbm, out_hbm, (e_pad, r, acc_rows) in zip(
            (i1, i2, i3, i4, i5), (o1, o2, o3, o4, o5), _HIST_CFG):
        base = c * r
        dmask = acc_rows - r - 1
        n_ch = e_pad // NTILE // CHUNK
        rpt = r // NTILE
        for z in range(acc_rows // NTILE // CHUNK):
            pltpu.sync_copy(zbuf, acc.at[pl.ds(s * (acc_rows // NTILE) + z * CHUNK, CHUNK)])
        plsc.subcore_barrier()

        rbase = s * n_ch

        def body(g, carry):
            row0 = rbase + g * NBUF
            pltpu.async_copy(idx_hbm.at[pl.ds(row0, NBUF)], dblk, gsem).wait()
            for b in range(NBUF):
                for j in range(CHUNK // LANES):
                    dv = dblk[b, pl.ds(j * LANES, LANES)]
                    loc = dv - base
                    ok = (loc >= 0) & (loc < r)
                    spread = (iota + ((g * NBUF + b) * CHUNK + j * LANES)) & dmask
                    sblk[b, pl.ds(j * LANES, LANES)] = jnp.where(ok, loc, r + spread)
            sd = [pltpu.async_copy(ones, acc.at[sblk.at[b]], ssem, add=True)
                  for b in range(NBUF)]
            for d in sd:
                d.wait()
            return carry

        lax.fori_loop(0, n_ch // NBUF, body, 0)
        plsc.subcore_barrier()
        pltpu.sync_copy(acc.at[pl.ds(s * rpt, rpt)], out_hbm.at[pl.ds(base + s * rpt, rpt)])
        plsc.subcore_barrier()


@functools.partial(
    pl.kernel,
    out_type=(jax.ShapeDtypeStruct((4096, D), jnp.float32),
              jax.ShapeDtypeStruct((8192, D), jnp.float32)),
    mesh=_mesh,
    compiler_params=pltpu.CompilerParams(use_tc_tiling_on_sc=False),
    scratch_types=[
        pltpu.VMEM((CHUNK,), jnp.int32),
        pltpu.VMEM((CHUNK, D), jnp.float32),
        pltpu.VMEM((CHUNK, D), jnp.float32),
        pltpu.SemaphoreType.DMA,
    ],
)
def _batch_gather(ilu, blu, uidx, ilb, blb, bidx, u_out, b_out,
                  ibuf, r1, r2, sem):
    c = lax.axis_index("c")
    s = lax.axis_index("s")
    wid = s * 2 + c

    def addrows(i, carry):
        for k in range(D // LANES):
            sl = pl.ds(k * LANES, LANES)
            r1[i, sl] = r1[i, sl] + r2[i, sl]
        return carry

    off = wid * CHUNK
    pltpu.sync_copy(uidx.at[pl.ds(off, CHUNK)], ibuf)
    pltpu.async_copy(ilu.at[ibuf], r1, sem).wait()
    pltpu.async_copy(blu.at[ibuf], r2, sem).wait()
    lax.fori_loop(0, CHUNK, addrows, 0)
    pltpu.sync_copy(r1, u_out.at[pl.ds(off, CHUNK)])

    for cc in range(2):
        off = wid * 2 * CHUNK + cc * CHUNK
        pltpu.sync_copy(bidx.at[pl.ds(off, CHUNK)], ibuf)
        pltpu.async_copy(ilb.at[ibuf], r1, sem).wait()
        pltpu.async_copy(blb.at[ibuf], r2, sem).wait()
        lax.fori_loop(0, CHUNK, addrows, 0)
        pltpu.sync_copy(r1, b_out.at[pl.ds(off, CHUNK)])


def _dinv_body(x_ref, o_ref):
    o_ref[...] = 1.0 / (jnp.sqrt(x_ref[...]) + 1e-8)


def _invdeg_body(x_ref, o_ref):
    o_ref[...] = 1.0 / (x_ref[...] + 1e-8)


def _ew(body, x):
    n = x.shape[0]
    x2 = x.reshape(n // 128, 128)
    out = pl.pallas_call(body, out_shape=jax.ShapeDtypeStruct(x2.shape, jnp.float32))(x2)
    return out.reshape(n)


def _rowscale_body(x_ref, s_ref, o_ref, *, square):
    sc = s_ref[...]
    if square:
        sc = sc * sc
    o_ref[...] = x_ref[...] * sc


def _rowscale(x, s, out_rows, square=False):
    n = x.shape[0]
    blk = 2000 if n % 2000 == 0 else 2048
    grid = n // blk
    return pl.pallas_call(
        functools.partial(_rowscale_body, square=square),
        grid=(grid,),
        in_specs=[pl.BlockSpec((blk, D), lambda i: (i, 0)),
                  pl.BlockSpec((blk, 1), lambda i: (i, 0))],
        out_specs=pl.BlockSpec((blk, D), lambda i: (i, 0)),
        out_shape=jax.ShapeDtypeStruct((out_rows, D), jnp.float32),
    )(x, s[:, None])


def _lrelu(x):
    return jnp.where(x > 0, x, 0.05 * x)


def _fuse_body(x_ref, r1_ref, r2_ref, s_ref, h_ref, o_ref):
    acc = (x_ref[...] + s_ref[...] * (r1_ref[...] + r2_ref[...])) * (1.0 / 3.0)
    t = _lrelu(jnp.dot(acc, h_ref[...], preferred_element_type=jnp.float32))
    t2 = _lrelu(lax.dot_general(t, h_ref[...], (((1,), (1,)), ((), ())),
                                preferred_element_type=jnp.float32))
    o_ref[...] = acc + t2


def _fuse(x, r1, r2, s, h, out_rows):
    n = x.shape[0]
    blk = 2000
    grid = n // blk
    return pl.pallas_call(
        _fuse_body,
        grid=(grid,),
        in_specs=[pl.BlockSpec((blk, D), lambda i: (i, 0)),
                  pl.BlockSpec((blk, D), lambda i: (i, 0)),
                  pl.BlockSpec((blk, D), lambda i: (i, 0)),
                  pl.BlockSpec((blk, 1), lambda i: (i, 0)),
                  pl.BlockSpec((D, H), lambda i: (0, 0))],
        out_specs=pl.BlockSpec((blk, D), lambda i: (i, 0)),
        out_shape=jax.ShapeDtypeStruct((out_rows, D), jnp.float32),
    )(x, r1, r2, s[:, None], h)


def _score_body(u_ref, b_ref, o_ref):
    u = u_ref[...]
    br = b_ref[...]
    p0 = jnp.sum(u * br[:, :D], axis=1, keepdims=True)
    p1 = jnp.sum(u * br[:, D:], axis=1, keepdims=True)
    o_ref[...] = jnp.concatenate([p0, p1], axis=1)


def _pad_scatter(idx, e_pad):
    pad = jnp.full((e_pad - idx.shape[0],), 100_000_000, jnp.int32)
    return jnp.concatenate([idx.astype(jnp.int32), pad]).reshape(-1, CHUNK)


def _pad_gather(idx, e_pad, n):
    pad = jnp.arange(e_pad - idx.shape[0], dtype=jnp.int32) % n
    return jnp.concatenate([idx.astype(jnp.int32), pad]).reshape(-1, CHUNK)


_spmm_a = _make_spmm_bin(2 * R50, EP8, R50, A50)
_spmm_b = _make_spmm_bin(2 * R20, EP8, R50, A50)
_spmm_c = _make_spmm_bin(2 * R50, EP8, R20, A20)
_spmm_d = _make_spmm_bin(2 * R50, EPB, R20, A20)


def kernel(users_feature, items_feature, bundles_feature, IL_user_hyper, IL_item_hyper, BL_user_hyper, BL_bundle_hyper, ui_src, ui_dst, ub_src, ub_dst, bi_src, bi_dst, users, bundles):
    ui_s_sc = _pad_scatter(ui_src, EP8)
    ui_s_ga = _pad_gather(ui_src, EP8, NU)
    ui_d_sc = _pad_scatter(ui_dst, EP8)
    ui_d_ga = _pad_gather(ui_dst, EP8, NI)
    ub_s_sc = _pad_scatter(ub_src, EP8)
    ub_s_ga = _pad_gather(ub_src, EP8, NU)
    ub_d_sc = _pad_scatter(ub_dst, EP8)
    ub_d_ga = _pad_gather(ub_dst, EP8, NB)
    bi_s_sc = _pad_scatter(bi_src, EPB)
    bi_d_ga = _pad_gather(bi_dst, EPB, NI)

    deg_ui_u, deg_ui_i, deg_ub_u, deg_ub_b, deg_bi = _hist5(
        ui_s_sc, ui_d_sc, ub_s_sc, ub_d_sc, bi_s_sc)

    (bgA, bsA, cA, bgB, bsB, cB, bgC, bsC, cC,
     bgD, bsD, cD, bgE, bsE, cE) = _bin5(
        ui_d_ga.reshape(-1), ui_s_sc.reshape(-1),
        ui_s_ga.reshape(-1), ui_d_sc.reshape(-1),
        ub_d_ga.reshape(-1), ub_s_sc.reshape(-1),
        ub_s_ga.reshape(-1), ub_d_sc.reshape(-1),
        bi_d_ga.reshape(-1), bi_s_sc.reshape(-1))

    ra_ui = _ew(_dinv_body, deg_ui_u)
    rb_ui = _ew(_dinv_body, deg_ui_i)
    ra_ub = _ew(_dinv_body, deg_ub_u)
    rb_ub = _ew(_dinv_body, deg_ub_b)
    inv_bi = _ew(_invdeg_body, deg_bi)

    yu_ui = _rowscale(users_feature, ra_ui[:NU], 2 * R50)
    yi_ui = _rowscale(items_feature, rb_ui[:NI], 2 * R50)
    yu_ub = _rowscale(users_feature, ra_ub[:NU], 2 * R50)
    yb_ub = _rowscale(bundles_feature, rb_ub[:NB], 2 * R20)

    raw_u1_il = _spmm_a(yi_ui, bgA, bsA, cA)
    raw_i1 = _spmm_a(yu_ui, bgB, bsB, cB)
    raw_u1_bl = _spmm_b(yb_ub, bgC, bsC, cC)
    raw_b1 = _spmm_c(yu_ub, bgD, bsD, cD)

    y2_i = _rowscale(raw_i1, rb_ui, 2 * R50, square=True)
    y2_u_il = _rowscale(raw_u1_il, ra_ui, 2 * R50, square=True)
    y2_b = _rowscale(raw_b1, rb_ub, 2 * R20, square=True)
    y2_u_bl = _rowscale(raw_u1_bl, ra_ub, 2 * R50, square=True)

    raw_u2_il = _spmm_a(y2_i, bgA, bsA, cA)
    raw_i2 = _spmm_a(y2_u_il, bgB, bsB, cB)
    raw_u2_bl = _spmm_b(y2_b, bgC, bsC, cC)
    raw_b2 = _spmm_c(y2_u_bl, bgD, bsD, cD)

    IL_users_f = _fuse(users_feature, raw_u1_il[:NU], raw_u2_il[:NU], ra_ui[:NU], IL_user_hyper, NU)
    IL_items_f = _fuse(items_feature, raw_i1[:NI], raw_i2[:NI], rb_ui[:NI], IL_item_hyper, 2 * R50)
    BL_users_f = _fuse(users_feature, raw_u1_bl[:NU], raw_u2_bl[:NU], ra_ub[:NU], BL_user_hyper, NU)
    BL_bundles_f = _fuse(bundles_feature, raw_b1[:NB], raw_b2[:NB], rb_ub[:NB], BL_bundle_hyper, NB)

    raw_bi = _spmm_d(IL_items_f, bgE, bsE, cE)
    il_bundles = _rowscale(raw_bi, inv_bi, 2 * R20)

    u_rows, b_rows = _batch_gather(
        IL_users_f, BL_users_f, users.astype(jnp.int32),
        il_bundles, BL_bundles_f, bundles.reshape(-1).astype(jnp.int32))

    pred = pl.pallas_call(
        _score_body,
        out_shape=jax.ShapeDtypeStruct((4096, 2), jnp.float32),
    )(u_rows, b_rows.reshape(4096, 2 * D))
    return pred


# speedup vs baseline: 26.9878x; 1.0001x over previous
import functools

import jax
import jax.numpy as jnp
from jax import lax
from jax.experimental import pallas as pl
from jax.experimental.pallas import tpu as pltpu
from jax.experimental.pallas import tpu_sc as plsc

NU, NI, NB, D, H = 50000, 50000, 20000, 64, 128
LANES = 16
CHUNK = 128
NTILE = 16

R50, A50 = 25600, 26624
R20, A20 = 10240, 12288
NBUF = 8
EP8 = 819200
EPB = 655360

_mesh = plsc.VectorSubcoreMesh(core_axis_name="c", subcore_axis_name="s")


_SENT = 100_000_000
_BIN_CFG = [(EP8, R50), (EP8, R50), (EP8, R50), (EP8, R20), (EPB, R20)]


def _bin_rc(e_pad):
    return e_pad // 32 // CHUNK + 2


_BIN_OUT = []
for _ep, _r in _BIN_CFG:
    _BIN_OUT += [jax.ShapeDtypeStruct((64 * _bin_rc(_ep) * CHUNK,), jnp.int32),
                 jax.ShapeDtypeStruct((64 * _bin_rc(_ep) * CHUNK,), jnp.int32),
                 jax.ShapeDtypeStruct((512,), jnp.int32)]


@functools.partial(
    pl.kernel,
    out_type=tuple(_BIN_OUT),
    mesh=_mesh,
    compiler_params=pltpu.CompilerParams(use_tc_tiling_on_sc=False,
                                         needs_layout_passes=False),
    scratch_types=[
        pltpu.VMEM((8 * CHUNK,), jnp.int32),
        pltpu.VMEM((8 * CHUNK,), jnp.int32),
        pltpu.VMEM((272,), jnp.int32),
        pltpu.VMEM((272,), jnp.int32),
        pltpu.VMEM((272,), jnp.int32),
        pltpu.VMEM((272,), jnp.int32),
        pltpu.VMEM((CHUNK,), jnp.int32),
        pltpu.VMEM((CHUNK,), jnp.int32),
        pltpu.VMEM((LANES,), jnp.int32),
        pltpu.SemaphoreType.DMA,
    ],
)
def _bin5(g1, d1, g2, d2, g3, d3, g4, d4, g5, d5,
          bg1, bs1, c1, bg2, bs2, c2, bg3, bs3, c3, bg4, bs4, c4, bg5, bs5, c5,
          gblk, dblk, bg0, bs0, bgh1, bsh1, padg, pads, cbuf, sem):
    c = lax.axis_index("c")
    s = lax.axis_index("s")
    w = s * 2 + c
    iota = lax.iota(jnp.int32, LANES)
    for j in range(CHUNK // LANES):
        padg[pl.ds(j * LANES, LANES)] = (iota + j * LANES) & 127
        pads[pl.ds(j * LANES, LANES)] = jnp.full((LANES,), _SENT, jnp.int32)

    for (gidx, didx, bgo, bso, co, (e_pad, r)) in (
            (g1, d1, bg1, bs1, c1, _BIN_CFG[0]),
            (g2, d2, bg2, bs2, c2, _BIN_CFG[1]),
            (g3, d3, bg3, bs3, c3, _BIN_CFG[2]),
            (g4, d4, bg4, bs4, c4, _BIN_CFG[3]),
            (g5, d5, bg5, bs5, c5, _BIN_CFG[4])):
        rpp = e_pad // 32 // CHUNK
        rc = rpp + 2
        in_off = w * rpp * CHUNK

        def chunk_body(ch, carry, bgo=bgo, bso=bso, r=r, rc=rc):
            offs = [carry[0], carry[2]]
            fs = [carry[1], carry[3]]
            for h, (bg, bs) in enumerate(((bg0, bs0), (bgh1, bsh1))):
                lo = h * r
                for j in range(CHUNK // LANES):
                    d16 = dblk[pl.ds(ch * CHUNK + j * LANES, LANES)]
                    g16 = gblk[pl.ds(ch * CHUNK + j * LANES, LANES)]
                    m = (d16 >= lo) & (d16 < lo + r)
                    plsc.store_compressed(bg.at[pl.ds(offs[h], LANES)], g16, mask=m)
                    plsc.store_compressed(bs.at[pl.ds(offs[h], LANES)], d16, mask=m)
                    cum = plsc.cumsum(jnp.where(m, 1, 0).astype(jnp.int32))
                    offs[h] = offs[h] + cum[15]
                do = offs[h] >= CHUNK
                orow = (h * 32 + w) * rc + fs[h]

                @pl.when(do)
                def _(bg=bg, bs=bs, orow=orow, bgo=bgo, bso=bso):
                    pltpu.sync_copy(bg.at[pl.ds(0, CHUNK)], bgo.at[pl.ds(orow * CHUNK, CHUNK)])
                    pltpu.sync_copy(bs.at[pl.ds(0, CHUNK)], bso.at[pl.ds(orow * CHUNK, CHUNK)])
                    for grp in range(CHUNK // LANES):
                        sl = pl.ds(grp * LANES, LANES)
                        sh = pl.ds(CHUNK + grp * LANES, LANES)
                        bg[sl] = bg[sh]
                        bs[sl] = bs[sh]

                offs[h] = jnp.where(do, offs[h] - CHUNK, offs[h])
                fs[h] = jnp.where(do, fs[h] + 1, fs[h])
            return offs[0], fs[0], offs[1], fs[1]

        def super_body(gsup, carry):
            off0 = in_off + gsup * 8 * CHUNK
            dd1 = pltpu.async_copy(gidx.at[pl.ds(off0, 8 * CHUNK)], gblk, sem)
            dd2 = pltpu.async_copy(didx.at[pl.ds(off0, 8 * CHUNK)], dblk, sem)
            dd1.wait()
            dd2.wait()
            return lax.fori_loop(0, 8, chunk_body, carry)

        off0, f0, off1, f1 = lax.fori_loop(0, rpp // 8, super_body,
                                           (jnp.int32(0),) * 4)

        nchs = []
        for h, (bg, bs), off, f in ((0, (bg0, bs0), off0, f0),
                                    (1, (bgh1, bsh1), off1, f1)):
            for j in range(CHUNK // LANES):
                bg[pl.ds(off + j * LANES, LANES)] = (iota + j * LANES) & 127
                bs[pl.ds(off + j * LANES, LANES)] = jnp.full((LANES,), _SENT, jnp.int32)
            t1 = f + (off > 0).astype(jnp.int32)
            orow = (h * 32 + w) * rc

            @pl.when(off > 0)
            def _(bg=bg, bs=bs, orow=orow, f=f, bgo=bgo, bso=bso):
                pltpu.sync_copy(bg.at[pl.ds(0, CHUNK)], bgo.at[pl.ds((orow + f) * CHUNK, CHUNK)])
                pltpu.sync_copy(bs.at[pl.ds(0, CHUNK)], bso.at[pl.ds((orow + f) * CHUNK, CHUNK)])

            @pl.when(t1 % 2 == 1)
            def _(orow=orow, t1=t1, bgo=bgo, bso=bso):
                pltpu.sync_copy(padg, bgo.at[pl.ds((orow + t1) * CHUNK, CHUNK)])
                pltpu.sync_copy(pads, bso.at[pl.ds((orow + t1) * CHUNK, CHUNK)])

            nchs.append(t1 + t1 % 2)

        cvec = jnp.where(iota == 0, nchs[0],
                         jnp.where(iota == 1, nchs[1], 0)).astype(jnp.int32)
        cbuf[...] = cvec
        pltpu.sync_copy(cbuf, co.at[pl.ds(w * LANES, LANES)])


def _make_spmm_bin(n_gather, e_pad, r, acc_rows):
    rc = _bin_rc(e_pad)
    rpt = r // NTILE
    zpt = acc_rows // NTILE // CHUNK
    dmask = acc_rows - r - 1

    @functools.partial(
        pl.kernel,
        out_type=jax.ShapeDtypeStruct((2 * r, D), jnp.float32),
        mesh=_mesh,
        compiler_params=pltpu.CompilerParams(use_tc_tiling_on_sc=False),
        scratch_types=[
            pltpu.VMEM_SHARED((acc_rows, D), jnp.float32),
            pltpu.VMEM((8 * CHUNK,), jnp.int32),
            pltpu.VMEM((8 * CHUNK,), jnp.int32),
            pltpu.VMEM((2, CHUNK), jnp.int32),
            pltpu.VMEM((2, CHUNK, D), jnp.float32),
            pltpu.VMEM((2 * LANES,), jnp.int32),
            pltpu.SemaphoreType.DMA,
            pltpu.SemaphoreType.DMA,
        ],
    )
    def spmm(y_hbm, bg_hbm, bs_hbm, cnt_hbm, out_hbm,
             acc, gblk, dblk, sblk, rbuf, cvbuf, gsem, ssem):
        c = lax.axis_index("c")
        s = lax.axis_index("s")
        base = c * r
        iota = lax.iota(jnp.int32, LANES)

        def zrow(i, carry):
            for k in range(D // LANES):
                rbuf[0, i, pl.ds(k * LANES, LANES)] = jnp.zeros((LANES,), jnp.float32)
            return carry

        lax.fori_loop(0, CHUNK, zrow, 0)
        for z in range(zpt):
            pltpu.sync_copy(rbuf.at[0],
                            acc.at[pl.ds(s * (acc_rows // NTILE) + z * CHUNK, CHUNK)])
        plsc.subcore_barrier()

        pltpu.sync_copy(cnt_hbm.at[pl.ds(s * 2 * LANES, 2 * LANES)], cvbuf)
        v0 = cvbuf[pl.ds(0, LANES)]
        v1 = cvbuf[pl.ds(LANES, LANES)]
        n0 = jnp.where(c == 0, v0[0], v0[1])
        n1 = jnp.where(c == 0, v1[0], v1[1])

        for t, nch in ((0, n0), (1, n1)):
            reg = (c * 32 + (2 * s + t)) * rc

            def pair(goff, ioff):
                gd = [pltpu.async_copy(
                    y_hbm.at[gblk.at[pl.ds((ioff + b) * CHUNK, CHUNK)]],
                    rbuf.at[b], gsem) for b in range(2)]
                for b in range(2):
                    for j in range(CHUNK // LANES):
                        dv = dblk[pl.ds((ioff + b) * CHUNK + j * LANES, LANES)]
                        loc = dv - base
                        ok = (loc >= 0) & (loc < r)
                        spread = (iota + ((goff + b) * CHUNK + j * LANES)) & dmask
                        sblk[b, pl.ds(j * LANES, LANES)] = jnp.where(ok, loc, r + spread)
                sd = []
                for b in range(2):
                    gd[b].wait()
                    sd.append(pltpu.async_copy(rbuf.at[b], acc.at[sblk.at[b]],
                                               ssem, add=True))
                for d in sd:
                    d.wait()

            def body8(g, carry):
                off0 = (reg + g * 8) * CHUNK
                dd1 = pltpu.async_copy(bg_hbm.at[pl.ds(off0, 8 * CHUNK)], gblk, gsem)
                dd2 = pltpu.async_copy(bs_hbm.at[pl.ds(off0, 8 * CHUNK)], dblk, gsem)
                dd1.wait()
                dd2.wait()
                for p in range(4):
                    pair(g * 8 + p * 2, p * 2)
                return carry

            def body2(g, carry):
                off0 = (reg + g * 2) * CHUNK
                dd1 = pltpu.async_copy(bg_hbm.at[pl.ds(off0, 2 * CHUNK)],
                                       gblk.at[pl.ds(0, 2 * CHUNK)], gsem)
                dd2 = pltpu.async_copy(bs_hbm.at[pl.ds(off0, 2 * CHUNK)],
                                       dblk.at[pl.ds(0, 2 * CHUNK)], gsem)
                dd1.wait()
                dd2.wait()
                pair(g * 2, 0)
                return carry

            n8 = nch // 8
            lax.fori_loop(0, n8, body8, 0)
            lax.fori_loop(n8 * 4, nch // 2, body2, 0)

        plsc.subcore_barrier()
        pltpu.sync_copy(acc.at[pl.ds(s * rpt, rpt)], out_hbm.at[pl.ds(base + s * rpt, rpt)])

    return spmm


_HIST_CFG = [(EP8, R50, A50), (EP8, R50, A50), (EP8, R50, A50),
             (EP8, R20, A20), (EPB, R20, A20)]


@functools.partial(
    pl.kernel,
    out_type=tuple(jax.ShapeDtypeStruct((2 * r,), jnp.float32) for (_, r, _) in _HIST_CFG),
    mesh=_mesh,
    compiler_params=pltpu.CompilerParams(use_tc_tiling_on_sc=False),
    scratch_types=[
        pltpu.VMEM_SHARED((A50,), jnp.float32),
        pltpu.VMEM((NBUF, CHUNK), jnp.int32),
        pltpu.VMEM((NBUF, CHUNK), jnp.int32),
        pltpu.VMEM((CHUNK,), jnp.float32),
        pltpu.VMEM((CHUNK,), jnp.float32),
        pltpu.SemaphoreType.DMA,
        pltpu.SemaphoreType.DMA,
    ],
)
def _hist5(i1, i2, i3, i4, i5, o1, o2, o3, o4, o5, acc, dblk, sblk, ones, zbuf, gsem, ssem):
    c = lax.axis_index("c")
    s = lax.axis_index("s")
    iota = lax.iota(jnp.int32, LANES)
    for j in range(CHUNK // LANES):
        ones[pl.ds(j * LANES, LANES)] = jnp.full((LANES,), 1.0, jnp.float32)
        zbuf[pl.ds(j * LANES, LANES)] = jnp.zeros((LANES,), jnp.float32)

    for idx_hbm, out_hbm, (e_pad, r, acc_rows) in zip(
            (i1, i2, i3, i4, i5), (o1, o2, o3, o4, o5), _HIST_CFG):
        base = c * r
        dmask = acc_rows - r - 1
        n_ch = e_pad // NTILE // CHUNK
        rpt = r // NTILE
        for z in range(acc_rows // NTILE // CHUNK):
            pltpu.sync_copy(zbuf, acc.at[pl.ds(s * (acc_rows // NTILE) + z * CHUNK, CHUNK)])
        plsc.subcore_barrier()

        rbase = s * n_ch

        def body(g, carry):
            row0 = rbase + g * NBUF
            pltpu.async_copy(idx_hbm.at[pl.ds(row0, NBUF)], dblk, gsem).wait()
            for b in range(NBUF):
                for j in range(CHUNK // LANES):
                    dv = dblk[b, pl.ds(j * LANES, LANES)]
                    loc = dv - base
                    ok = (loc >= 0) & (loc < r)
                    spread = (iota + ((g * NBUF + b) * CHUNK + j * LANES)) & dmask
                    sblk[b, pl.ds(j * LANES, LANES)] = jnp.where(ok, loc, r + spread)
            sd = [pltpu.async_copy(ones, acc.at[sblk.at[b]], ssem, add=True)
                  for b in range(NBUF)]
            for d in sd:
                d.wait()
            return carry

        lax.fori_loop(0, n_ch // NBUF, body, 0)
        plsc.subcore_barrier()
        pltpu.sync_copy(acc.at[pl.ds(s * rpt, rpt)], out_hbm.at[pl.ds(base + s * rpt, rpt)])
        plsc.subcore_barrier()


@functools.partial(
    pl.kernel,
    out_type=(jax.ShapeDtypeStruct((4096, D), jnp.float32),
              jax.ShapeDtypeStruct((8192, D), jnp.float32)),
    mesh=_mesh,
    compiler_params=pltpu.CompilerParams(use_tc_tiling_on_sc=False),
    scratch_types=[
        pltpu.VMEM((CHUNK,), jnp.int32),
        pltpu.VMEM((CHUNK, D), jnp.float32),
        pltpu.VMEM((CHUNK, D), jnp.float32),
        pltpu.SemaphoreType.DMA,
    ],
)
def _batch_gather(ilu, blu, uidx, ilb, blb, bidx, u_out, b_out,
                  ibuf, r1, r2, sem):
    c = lax.axis_index("c")
    s = lax.axis_index("s")
    wid = s * 2 + c

    def addrows(i, carry):
        for k in range(D // LANES):
            sl = pl.ds(k * LANES, LANES)
            r1[i, sl] = r1[i, sl] + r2[i, sl]
        return carry

    off = wid * CHUNK
    pltpu.sync_copy(uidx.at[pl.ds(off, CHUNK)], ibuf)
    pltpu.async_copy(ilu.at[ibuf], r1, sem).wait()
    pltpu.async_copy(blu.at[ibuf], r2, sem).wait()
    lax.fori_loop(0, CHUNK, addrows, 0)
    pltpu.sync_copy(r1, u_out.at[pl.ds(off, CHUNK)])

    for cc in range(2):
        off = wid * 2 * CHUNK + cc * CHUNK
        pltpu.sync_copy(bidx.at[pl.ds(off, CHUNK)], ibuf)
        pltpu.async_copy(ilb.at[ibuf], r1, sem).wait()
        pltpu.async_copy(blb.at[ibuf], r2, sem).wait()
        lax.fori_loop(0, CHUNK, addrows, 0)
        pltpu.sync_copy(r1, b_out.at[pl.ds(off, CHUNK)])


def _dinv_body(x_ref, o_ref):
    o_ref[...] = 1.0 / (jnp.sqrt(x_ref[...]) + 1e-8)


def _invdeg_body(x_ref, o_ref):
    o_ref[...] = 1.0 / (x_ref[...] + 1e-8)


def _ew(body, x):
    n = x.shape[0]
    x2 = x.reshape(n // 128, 128)
    out = pl.pallas_call(body, out_shape=jax.ShapeDtypeStruct(x2.shape, jnp.float32))(x2)
    return out.reshape(n)


def _rowscale_body(x_ref, s_ref, o_ref, *, square):
    sc = s_ref[...]
    if square:
        sc = sc * sc
    o_ref[...] = x_ref[...] * sc


def _rowscale(x, s, out_rows, square=False):
    n = x.shape[0]
    blk = 2000 if n % 2000 == 0 else 2048
    grid = n // blk
    return pl.pallas_call(
        functools.partial(_rowscale_body, square=square),
        grid=(grid,),
        in_specs=[pl.BlockSpec((blk, D), lambda i: (i, 0)),
                  pl.BlockSpec((blk, 1), lambda i: (i, 0))],
        out_specs=pl.BlockSpec((blk, D), lambda i: (i, 0)),
        out_shape=jax.ShapeDtypeStruct((out_rows, D), jnp.float32),
    )(x, s[:, None])


def _lrelu(x):
    return jnp.where(x > 0, x, 0.05 * x)


def _fuse_body(x_ref, r1_ref, r2_ref, s_ref, h_ref, o_ref):
    acc = (x_ref[...] + s_ref[...] * (r1_ref[...] + r2_ref[...])) * (1.0 / 3.0)
    t = _lrelu(jnp.dot(acc, h_ref[...], preferred_element_type=jnp.float32))
    t2 = _lrelu(lax.dot_general(t, h_ref[...], (((1,), (1,)), ((), ())),
                                preferred_element_type=jnp.float32))
    o_ref[...] = acc + t2


def _fuse(x, r1, r2, s, h, out_rows):
    n = x.shape[0]
    blk = 2000
    grid = n // blk
    return pl.pallas_call(
        _fuse_body,
        grid=(grid,),
        in_specs=[pl.BlockSpec((blk, D), lambda i: (i, 0)),
                  pl.BlockSpec((blk, D), lambda i: (i, 0)),
                  pl.BlockSpec((blk, D), lambda i: (i, 0)),
                  pl.BlockSpec((blk, 1), lambda i: (i, 0)),
                  pl.BlockSpec((D, H), lambda i: (0, 0))],
        out_specs=pl.BlockSpec((blk, D), lambda i: (i, 0)),
        out_shape=jax.ShapeDtypeStruct((out_rows, D), jnp.float32),
    )(x, r1, r2, s[:, None], h)


def _score_body(u_ref, b_ref, o_ref):
    u = u_ref[...]
    br = b_ref[...]
    p0 = jnp.sum(u * br[:, :D], axis=1, keepdims=True)
    p1 = jnp.sum(u * br[:, D:], axis=1, keepdims=True)
    o_ref[...] = jnp.concatenate([p0, p1], axis=1)


def _pad_scatter(idx, e_pad):
    pad = jnp.full((e_pad - idx.shape[0],), 100_000_000, jnp.int32)
    return jnp.concatenate([idx.astype(jnp.int32), pad]).reshape(-1, CHUNK)


def _pad_gather(idx, e_pad, n):
    pad = jnp.arange(e_pad - idx.shape[0], dtype=jnp.int32) % n
    return jnp.concatenate([idx.astype(jnp.int32), pad]).reshape(-1, CHUNK)


_spmm_a = _make_spmm_bin(2 * R50, EP8, R50, A50)
_spmm_b = _make_spmm_bin(2 * R20, EP8, R50, A50)
_spmm_c = _make_spmm_bin(2 * R50, EP8, R20, A20)
_spmm_d = _make_spmm_bin(2 * R50, EPB, R20, A20)


def kernel(users_feature, items_feature, bundles_feature, IL_user_hyper, IL_item_hyper, BL_user_hyper, BL_bundle_hyper, ui_src, ui_dst, ub_src, ub_dst, bi_src, bi_dst, users, bundles):
    ui_s_sc = _pad_scatter(ui_src, EP8)
    ui_s_ga = _pad_gather(ui_src, EP8, NU)
    ui_d_sc = _pad_scatter(ui_dst, EP8)
    ui_d_ga = _pad_gather(ui_dst, EP8, NI)
    ub_s_sc = _pad_scatter(ub_src, EP8)
    ub_s_ga = _pad_gather(ub_src, EP8, NU)
    ub_d_sc = _pad_scatter(ub_dst, EP8)
    ub_d_ga = _pad_gather(ub_dst, EP8, NB)
    bi_s_sc = _pad_scatter(bi_src, EPB)
    bi_d_ga = _pad_gather(bi_dst, EPB, NI)

    deg_ui_u, deg_ui_i, deg_ub_u, deg_ub_b, deg_bi = _hist5(
        ui_s_sc, ui_d_sc, ub_s_sc, ub_d_sc, bi_s_sc)

    (bgA, bsA, cA, bgB, bsB, cB, bgC, bsC, cC,
     bgD, bsD, cD, bgE, bsE, cE) = _bin5(
        ui_d_ga.reshape(-1), ui_s_sc.reshape(-1),
        ui_s_ga.reshape(-1), ui_d_sc.reshape(-1),
        ub_d_ga.reshape(-1), ub_s_sc.reshape(-1),
        ub_s_ga.reshape(-1), ub_d_sc.reshape(-1),
        bi_d_ga.reshape(-1), bi_s_sc.reshape(-1))

    ra_ui = _ew(_dinv_body, deg_ui_u)
    rb_ui = _ew(_dinv_body, deg_ui_i)
    ra_ub = _ew(_dinv_body, deg_ub_u)
    rb_ub = _ew(_dinv_body, deg_ub_b)
    inv_bi = _ew(_invdeg_body, deg_bi)

    yu_ui = _rowscale(users_feature, ra_ui[:NU], 2 * R50)
    yi_ui = _rowscale(items_feature, rb_ui[:NI], 2 * R50)
    yu_ub = _rowscale(users_feature, ra_ub[:NU], 2 * R50)
    yb_ub = _rowscale(bundles_feature, rb_ub[:NB], 2 * R20)

    raw_u1_il = _spmm_a(yi_ui, bgA, bsA, cA)
    raw_i1 = _spmm_a(yu_ui, bgB, bsB, cB)
    raw_u1_bl = _spmm_b(yb_ub, bgC, bsC, cC)
    raw_b1 = _spmm_c(yu_ub, bgD, bsD, cD)

    y2_i = _rowscale(raw_i1, rb_ui, 2 * R50, square=True)
    y2_u_il = _rowscale(raw_u1_il, ra_ui, 2 * R50, square=True)
    y2_b = _rowscale(raw_b1, rb_ub, 2 * R20, square=True)
    y2_u_bl = _rowscale(raw_u1_bl, ra_ub, 2 * R50, square=True)

    raw_u2_il = _spmm_a(y2_i, bgA, bsA, cA)
    raw_i2 = _spmm_a(y2_u_il, bgB, bsB, cB)
    raw_u2_bl = _spmm_b(y2_b, bgC, bsC, cC)
    raw_b2 = _spmm_c(y2_u_bl, bgD, bsD, cD)

    IL_users_f = _fuse(users_feature, raw_u1_il[:NU], raw_u2_il[:NU], ra_ui[:NU], IL_user_hyper, NU)
    IL_items_f = _fuse(items_feature, raw_i1[:NI], raw_i2[:NI], rb_ui[:NI], IL_item_hyper, 2 * R50)
    BL_users_f = _fuse(users_feature, raw_u1_bl[:NU], raw_u2_bl[:NU], ra_ub[:NU], BL_user_hyper, NU)
    BL_bundles_f = _fuse(bundles_feature, raw_b1[:NB], raw_b2[:NB], rb_ub[:NB], BL_bundle_hyper, NB)

    raw_bi = _spmm_d(IL_items_f, bgE, bsE, cE)
    il_bundles = _rowscale(raw_bi, inv_bi, 2 * R20)

    u_rows, b_rows = _batch_gather(
        IL_users_f, BL_users_f, users.astype(jnp.int32),
        il_bundles, BL_bundles_f, bundles.reshape(-1).astype(jnp.int32))

    pred = pl.pallas_call(
        _score_body,
        out_shape=jax.ShapeDtypeStruct((4096, 2), jnp.float32),
    )(u_rows, b_rows.reshape(4096, 2 * D))
    return pred
